# baseline (speedup 1.0000x reference)
import functools

import jax
import jax.numpy as jnp
import numpy as np
from jax import lax
from jax.experimental import pallas as pl
from jax.experimental.pallas import tpu as pltpu
from jax.experimental.pallas import tpu_sc as plsc

HID = 256
NG = 64
NL = 4
CUT = 5.0
NMOL = 64
N_NODES = 10000
N_PAD = 10240
N_EDGES = 160000
LOG2 = float(np.log(2.0))

NC = 2
NS = 16
NW = NC * NS

GB = 128
E_PAD = 163840
GNB = E_PAD // (NW * GB)
SNB = E_PAD // (NS * GB)
NPT = N_PAD // NS
FH = HID // NC


def _ssp(x):
    return jnp.maximum(x, 0.0) + jnp.log1p(jnp.exp(-jnp.abs(x))) - LOG2


def _sc_gather(table, idx2d):
    mesh = plsc.VectorSubcoreMesh(core_axis_name="c", subcore_axis_name="s")

    @functools.partial(
        pl.kernel,
        mesh=mesh,
        out_type=jax.ShapeDtypeStruct((E_PAD, HID // 2), jnp.int32),
        scratch_types=[
            pltpu.VMEM((GNB, GB), jnp.int32),
            pltpu.VMEM((GB, HID // 2), jnp.int32),
            pltpu.VMEM((GB, HID // 2), jnp.int32),
            pltpu.SemaphoreType.DMA,
            pltpu.SemaphoreType.DMA,
            pltpu.SemaphoreType.DMA,
            pltpu.SemaphoreType.DMA,
        ],
    )
    def k(table_hbm, idx_hbm, out_hbm, idx_v, buf0, buf1, gs0, gs1, ws0, ws1):
        wid = lax.axis_index("s") * NC + lax.axis_index("c")
        blk0 = wid * GNB
        pltpu.sync_copy(idx_hbm.at[pl.ds(blk0, GNB)], idx_v)

        def body(j2, carry):
            j0 = 2 * j2
            j1 = j0 + 1

            @pl.when(j2 > 0)
            def _():
                pltpu.make_async_copy(
                    buf0, out_hbm.at[pl.ds(0, GB)], ws0).wait()

            g0 = pltpu.async_copy(table_hbm.at[idx_v.at[j0]], buf0, gs0)

            @pl.when(j2 > 0)
            def _():
                pltpu.make_async_copy(
                    buf1, out_hbm.at[pl.ds(0, GB)], ws1).wait()

            g1 = pltpu.async_copy(table_hbm.at[idx_v.at[j1]], buf1, gs1)
            g0.wait()
            pltpu.async_copy(buf0, out_hbm.at[pl.ds((blk0 + j0) * GB, GB)], ws0)
            g1.wait()
            pltpu.async_copy(buf1, out_hbm.at[pl.ds((blk0 + j1) * GB, GB)], ws1)
            return carry

        lax.fori_loop(0, GNB // 2, body, 0)
        pltpu.make_async_copy(buf0, out_hbm.at[pl.ds(0, GB)], ws0).wait()
        pltpu.make_async_copy(buf1, out_hbm.at[pl.ds(0, GB)], ws1).wait()

    return k(table, idx2d)


def _sc_scatter_add(msg, idx2d):
    mesh = plsc.VectorSubcoreMesh(core_axis_name="c", subcore_axis_name="s")

    @functools.partial(
        pl.kernel,
        mesh=mesh,
        out_type=jax.ShapeDtypeStruct((N_PAD, HID), jnp.float32),
        scratch_types=[
            pltpu.VMEM((SNB, GB), jnp.int32),
            pltpu.VMEM((GB, FH), jnp.float32),
            pltpu.VMEM((GB, FH), jnp.float32),
            pltpu.VMEM_SHARED((N_PAD, FH), jnp.float32),
            pltpu.SemaphoreType.DMA,
            pltpu.SemaphoreType.DMA,
            pltpu.SemaphoreType.DMA,
            pltpu.SemaphoreType.DMA,
        ],
    )
    def k(msg_hbm, idx_hbm, out_hbm, idx_v, mb0, mb1, acc, ls0, ls1, ss0, ss1):
        c = lax.axis_index("c")
        s = lax.axis_index("s")

        def zrow(i, carry):
            for kk in range(FH // 16):
                mb0[i, pl.ds(kk * 16, 16)] = jnp.zeros((16,), jnp.float32)
            return carry

        lax.fori_loop(0, GB, zrow, 0)
        for t in range(NPT // GB):
            pltpu.sync_copy(mb0, acc.at[pl.ds(s * NPT + t * GB, GB)])
        plsc.subcore_barrier()

        pltpu.sync_copy(idx_hbm.at[pl.ds(s * SNB, SNB)], idx_v)

        def body(j2, carry):
            j0 = 2 * j2
            j1 = j0 + 1

            @pl.when(j2 > 0)
            def _():
                pltpu.make_async_copy(mb0, acc.at[pl.ds(0, GB)], ss0).wait()

            l0 = pltpu.async_copy(
                msg_hbm.at[c, pl.ds((s * SNB + j0) * GB, GB)], mb0, ls0)

            @pl.when(j2 > 0)
            def _():
                pltpu.make_async_copy(mb1, acc.at[pl.ds(0, GB)], ss1).wait()

            l1 = pltpu.async_copy(
                msg_hbm.at[c, pl.ds((s * SNB + j1) * GB, GB)], mb1, ls1)
            l0.wait()
            pltpu.async_copy(mb0, acc.at[idx_v.at[j0]], ss0, add=True)
            l1.wait()
            pltpu.async_copy(mb1, acc.at[idx_v.at[j1]], ss1, add=True)
            return carry

        lax.fori_loop(0, SNB // 2, body, 0)
        pltpu.make_async_copy(mb0, acc.at[pl.ds(0, GB)], ss0).wait()
        pltpu.make_async_copy(mb1, acc.at[pl.ds(0, GB)], ss1).wait()
        plsc.subcore_barrier()
        pltpu.sync_copy(acc.at[pl.ds(s * NPT, NPT)],
                        out_hbm.at[pl.ds(s * NPT, NPT), pl.ds(c * FH, FH)])

    return k(msg, idx2d)


BE = 2048
BN = 1024


def _rbf_kernel(dist_c):
    coeff = -0.5 / (CUT / NG) ** 2
    step = CUT / (NG - 1)

    def body(d_ref, rbf_ref, cut_ref):
        d = d_ref[...]
        offset = lax.broadcasted_iota(
            jnp.int32, (1, NG), 1).astype(jnp.float32) * step
        diff = d - offset
        rbf_ref[...] = jnp.exp(coeff * diff * diff)
        cut_ref[...] = 0.5 * (jnp.cos(d * (np.pi / CUT)) + 1.0) * (
            d < CUT).astype(jnp.float32)

    return pl.pallas_call(
        body,
        grid=(E_PAD // BE,),
        in_specs=[pl.BlockSpec((BE, 1), lambda i: (i, 0))],
        out_specs=[pl.BlockSpec((BE, NG), lambda i: (i, 0)),
                   pl.BlockSpec((BE, 1), lambda i: (i, 0))],
        out_shape=[jax.ShapeDtypeStruct((E_PAD, NG), jnp.float32),
                   jax.ShapeDtypeStruct((E_PAD, 1), jnp.float32)],
    )(dist_c)


def _emb_kernel(z_c, emb_pad, w1t, b1):
    def body(z_ref, emb_ref, w_ref, b_ref, h_ref, hA_ref):
        onehot = (z_ref[...] == lax.broadcasted_iota(jnp.int32, (1, 128), 1)
                  ).astype(jnp.float32)
        h = jnp.dot(onehot, emb_ref[...], preferred_element_type=jnp.float32)
        h_ref[...] = h
        hA_ref[...] = (jnp.dot(h, w_ref[...], preferred_element_type=jnp.float32)
                       + b_ref[...]).astype(jnp.bfloat16)

    return pl.pallas_call(
        body,
        grid=(N_PAD // BN,),
        in_specs=[pl.BlockSpec((BN, 1), lambda i: (i, 0)),
                  pl.BlockSpec((128, HID), lambda i: (0, 0)),
                  pl.BlockSpec((HID, HID), lambda i: (0, 0)),
                  pl.BlockSpec((1, HID), lambda i: (0, 0))],
        out_specs=[pl.BlockSpec((BN, HID), lambda i: (i, 0)),
                   pl.BlockSpec((BN, HID), lambda i: (i, 0))],
        out_shape=[jax.ShapeDtypeStruct((N_PAD, HID), jnp.float32),
                   jax.ShapeDtypeStruct((N_PAD, HID), jnp.bfloat16)],
    )(z_c, emb_pad, w1t, b1)


def _filter_kernel(rbf, cut, g, w1t, b1, w2t, b2):
    def body(rbf_ref, cut_ref, g_ref, w1_ref, b1_ref, w2_ref, b2_ref, o_ref):
        t = jnp.dot(rbf_ref[...], w1_ref[...],
                    preferred_element_type=jnp.float32) + b1_ref[...]
        t = _ssp(t)
        w = jnp.dot(t, w2_ref[...],
                    preferred_element_type=jnp.float32) + b2_ref[...]
        m = g_ref[...].astype(jnp.float32) * w * cut_ref[...]
        o_ref[0] = m[:, :FH]
        o_ref[1] = m[:, FH:]

    return pl.pallas_call(
        body,
        grid=(E_PAD // BE,),
        in_specs=[pl.BlockSpec((BE, NG), lambda i: (i, 0)),
                  pl.BlockSpec((BE, 1), lambda i: (i, 0)),
                  pl.BlockSpec((BE, HID), lambda i: (i, 0)),
                  pl.BlockSpec((NG, HID), lambda i: (0, 0)),
                  pl.BlockSpec((1, HID), lambda i: (0, 0)),
                  pl.BlockSpec((HID, HID), lambda i: (0, 0)),
                  pl.BlockSpec((1, HID), lambda i: (0, 0))],
        out_specs=pl.BlockSpec((NC, BE, FH), lambda i: (0, i, 0)),
        out_shape=jax.ShapeDtypeStruct((NC, E_PAD, FH), jnp.float32),
    )(rbf, cut, g, w1t, b1, w2t, b2)


def _node_kernel(agg, h, w2t, b2, w3t, b3, wnt=None, bn=None):
    dual = wnt is not None

    def body(agg_ref, h_ref, w2_ref, b2_ref, w3_ref, b3_ref, *rest):
        if dual:
            wn_ref, bn_ref, hn_ref, hA_ref = rest
        else:
            (hn_ref,) = rest
        t = jnp.dot(agg_ref[...], w2_ref[...],
                    preferred_element_type=jnp.float32) + b2_ref[...]
        t = _ssp(t)
        t = jnp.dot(t, w3_ref[...],
                    preferred_element_type=jnp.float32) + b3_ref[...]
        hn = h_ref[...] + t
        hn_ref[...] = hn
        if dual:
            hA_ref[...] = (jnp.dot(hn, wn_ref[...],
                                   preferred_element_type=jnp.float32)
                           + bn_ref[...]).astype(jnp.bfloat16)

    full = lambda i: (0, 0)
    blk = lambda i: (i, 0)
    in_specs = [pl.BlockSpec((BN, HID), blk), pl.BlockSpec((BN, HID), blk),
                pl.BlockSpec((HID, HID), full), pl.BlockSpec((1, HID), full),
                pl.BlockSpec((HID, HID), full), pl.BlockSpec((1, HID), full)]
    args = [agg, h, w2t, b2, w3t, b3]
    if dual:
        in_specs += [pl.BlockSpec((HID, HID), full), pl.BlockSpec((1, HID), full)]
        args += [wnt, bn]
        out_specs = [pl.BlockSpec((BN, HID), blk), pl.BlockSpec((BN, HID), blk)]
        out_shape = [jax.ShapeDtypeStruct((N_PAD, HID), jnp.float32),
                     jax.ShapeDtypeStruct((N_PAD, HID), jnp.bfloat16)]
    else:
        out_specs = pl.BlockSpec((BN, HID), blk)
        out_shape = jax.ShapeDtypeStruct((N_PAD, HID), jnp.float32)

    return pl.pallas_call(
        body,
        grid=(N_PAD // BN,),
        in_specs=in_specs,
        out_specs=out_specs,
        out_shape=out_shape,
    )(*args)


def _readout_kernel(h, batch_c, r1wt, r1b, r2w, r2b):
    def body(h_ref, b_ref, w1_ref, b1_ref, w2_ref, b2_ref, o_ref):
        t = jnp.dot(h_ref[...], w1_ref[...],
                    preferred_element_type=jnp.float32) + b1_ref[...]
        t = _ssp(t)
        e = jnp.sum(t * w2_ref[...], axis=1, keepdims=True) + b2_ref[...]
        onehot = (b_ref[...] == lax.broadcasted_iota(jnp.int32, (1, NMOL), 1)
                  ).astype(jnp.float32)
        mol = jnp.sum(onehot * e, axis=0, keepdims=True)

        @pl.when(pl.program_id(0) == 0)
        def _():
            o_ref[...] = jnp.zeros_like(o_ref)

        o_ref[...] += mol

    return pl.pallas_call(
        body,
        grid=(N_PAD // BN,),
        in_specs=[pl.BlockSpec((BN, HID), lambda i: (i, 0)),
                  pl.BlockSpec((BN, 1), lambda i: (i, 0)),
                  pl.BlockSpec((HID, HID // 2), lambda i: (0, 0)),
                  pl.BlockSpec((1, HID // 2), lambda i: (0, 0)),
                  pl.BlockSpec((1, HID // 2), lambda i: (0, 0)),
                  pl.BlockSpec((1, 1), lambda i: (0, 0))],
        out_specs=pl.BlockSpec((1, NMOL), lambda i: (0, 0)),
        out_shape=jax.ShapeDtypeStruct((1, NMOL), jnp.float32),
    )(h, batch_c, r1wt, r1b, r2w, r2b)


def kernel(z, edge_index, edge_attr, batch, emb, fw1, fb1, fw2, fb2,
           a1w, a1b, a2w, a2b, a3w, a3b, r1w, r1b, r2w, r2b):
    row = edge_index[0]
    col = edge_index[1]
    dist = edge_attr[:, 0]

    pad = E_PAD - N_EDGES
    col2d = jnp.concatenate(
        [col, jnp.zeros((pad,), col.dtype)]).reshape(E_PAD // GB, GB)
    row2d = jnp.concatenate(
        [row, jnp.zeros((pad,), row.dtype)]).reshape(E_PAD // GB, GB)
    dist_c = jnp.concatenate(
        [dist, jnp.full((pad,), 2.0 * CUT, dist.dtype)]).reshape(E_PAD, 1)

    emb_pad = jnp.zeros((128, HID), jnp.float32).at[: emb.shape[0]].set(emb)
    npad = N_PAD - N_NODES
    z_c = jnp.concatenate(
        [z.astype(jnp.int32), jnp.zeros((npad,), jnp.int32)]).reshape(N_PAD, 1)
    batch_c = jnp.concatenate(
        [batch.astype(jnp.int32),
         jnp.full((npad,), NMOL, jnp.int32)]).reshape(N_PAD, 1)

    fw1t = jnp.swapaxes(fw1, 1, 2)
    fw2t = jnp.swapaxes(fw2, 1, 2)
    a1wt = jnp.swapaxes(a1w, 1, 2)
    a2wt = jnp.swapaxes(a2w, 1, 2)
    a3wt = jnp.swapaxes(a3w, 1, 2)
    fb1_2 = fb1[:, None, :]
    fb2_2 = fb2[:, None, :]
    a1b_2 = a1b[:, None, :]
    a2b_2 = a2b[:, None, :]
    a3b_2 = a3b[:, None, :]
    r1wt = r1w.T
    r1b_2 = r1b[None, :]
    r2b_2 = r2b[None, :]

    rbf, cut = _rbf_kernel(dist_c)
    h, hA = _emb_kernel(z_c, emb_pad, a1wt[0], a1b_2[0])

    for l in range(NL):
        hA_pack = lax.bitcast_convert_type(
            hA.reshape(N_PAD, HID // 2, 2), jnp.int32)
        g_pack = _sc_gather(hA_pack, col2d)
        g = lax.bitcast_convert_type(
            g_pack, jnp.bfloat16).reshape(E_PAD, HID)
        msg = _filter_kernel(rbf, cut, g, fw1t[l], fb1_2[l], fw2t[l], fb2_2[l])
        agg = _sc_scatter_add(msg, row2d)
        if l < NL - 1:
            h, hA = _node_kernel(agg, h, a2wt[l], a2b_2[l], a3wt[l], a3b_2[l],
                                 a1wt[l + 1], a1b_2[l + 1])
        else:
            h = _node_kernel(agg, h, a2wt[l], a2b_2[l], a3wt[l], a3b_2[l])

    mol = _readout_kernel(h, batch_c, r1wt, r1b_2, r2w, r2b_2)
    return mol[0]


# speedup vs baseline: 2.1524x; 2.1524x over previous
import functools

import jax
import jax.numpy as jnp
import numpy as np
from jax import lax
from jax.experimental import pallas as pl
from jax.experimental.pallas import tpu as pltpu
from jax.experimental.pallas import tpu_sc as plsc

HID = 256
NG = 64
NL = 4
CUT = 5.0
NMOL = 64
N_NODES = 10000
N_PAD = 10240
N_EDGES = 160000
LOG2 = float(np.log(2.0))

NC = 2
NS = 16
NW = NC * NS

GB = 128
E_PAD = 163840
GNB = E_PAD // (NW * GB)
SNB = E_PAD // (NS * GB)
NPT = N_PAD // NS
FH = HID // NC


def _ssp(x):
    return jnp.maximum(x, 0.0) + jnp.log1p(jnp.exp(-jnp.abs(x))) - LOG2


def _pack_bf16_pair(x):
    lo = lax.bitcast_convert_type(
        x[:, :FH].astype(jnp.bfloat16), jnp.uint16).astype(jnp.int32)
    hi = lax.bitcast_convert_type(
        x[:, FH:].astype(jnp.bfloat16), jnp.uint16).astype(jnp.int32)
    return (hi << 16) | lo


def _unpack_bf16_pair(p):
    lo = lax.bitcast_convert_type(p << 16, jnp.float32)
    hi = lax.bitcast_convert_type(
        p & jnp.int32(np.uint32(0xFFFF0000)), jnp.float32)
    return lo, hi


def _sc_gather(table, idx2d):
    mesh = plsc.VectorSubcoreMesh(core_axis_name="c", subcore_axis_name="s")

    @functools.partial(
        pl.kernel,
        mesh=mesh,
        out_type=jax.ShapeDtypeStruct((E_PAD, HID // 2), jnp.int32),
        scratch_types=[
            pltpu.VMEM((GNB, GB), jnp.int32),
            pltpu.VMEM((GB, HID // 2), jnp.int32),
            pltpu.VMEM((GB, HID // 2), jnp.int32),
            pltpu.SemaphoreType.DMA,
            pltpu.SemaphoreType.DMA,
            pltpu.SemaphoreType.DMA,
            pltpu.SemaphoreType.DMA,
        ],
    )
    def k(table_hbm, idx_hbm, out_hbm, idx_v, buf0, buf1, gs0, gs1, ws0, ws1):
        wid = lax.axis_index("s") * NC + lax.axis_index("c")
        blk0 = wid * GNB
        pltpu.sync_copy(idx_hbm.at[pl.ds(blk0, GNB)], idx_v)

        def body(j2, carry):
            j0 = 2 * j2
            j1 = j0 + 1

            @pl.when(j2 > 0)
            def _():
                pltpu.make_async_copy(
                    buf0, out_hbm.at[pl.ds(0, GB)], ws0).wait()

            g0 = pltpu.async_copy(table_hbm.at[idx_v.at[j0]], buf0, gs0)

            @pl.when(j2 > 0)
            def _():
                pltpu.make_async_copy(
                    buf1, out_hbm.at[pl.ds(0, GB)], ws1).wait()

            g1 = pltpu.async_copy(table_hbm.at[idx_v.at[j1]], buf1, gs1)
            g0.wait()
            pltpu.async_copy(buf0, out_hbm.at[pl.ds((blk0 + j0) * GB, GB)], ws0)
            g1.wait()
            pltpu.async_copy(buf1, out_hbm.at[pl.ds((blk0 + j1) * GB, GB)], ws1)
            return carry

        lax.fori_loop(0, GNB // 2, body, 0)
        pltpu.make_async_copy(buf0, out_hbm.at[pl.ds(0, GB)], ws0).wait()
        pltpu.make_async_copy(buf1, out_hbm.at[pl.ds(0, GB)], ws1).wait()

    return k(table, idx2d)


def _sc_scatter_add(msg, idx2d):
    mesh = plsc.VectorSubcoreMesh(core_axis_name="c", subcore_axis_name="s")

    @functools.partial(
        pl.kernel,
        mesh=mesh,
        out_type=jax.ShapeDtypeStruct((N_PAD, HID), jnp.float32),
        scratch_types=[
            pltpu.VMEM((SNB, GB), jnp.int32),
            pltpu.VMEM((GB, FH), jnp.float32),
            pltpu.VMEM((GB, FH), jnp.float32),
            pltpu.VMEM_SHARED((N_PAD, FH), jnp.float32),
            pltpu.SemaphoreType.DMA,
            pltpu.SemaphoreType.DMA,
            pltpu.SemaphoreType.DMA,
            pltpu.SemaphoreType.DMA,
        ],
    )
    def k(msg_hbm, idx_hbm, out_hbm, idx_v, mb0, mb1, acc, ls0, ls1, ss0, ss1):
        c = lax.axis_index("c")
        s = lax.axis_index("s")

        def zrow(i, carry):
            for kk in range(FH // 16):
                mb0[i, pl.ds(kk * 16, 16)] = jnp.zeros((16,), jnp.float32)
            return carry

        lax.fori_loop(0, GB, zrow, 0)
        for t in range(NPT // GB):
            pltpu.sync_copy(mb0, acc.at[pl.ds(s * NPT + t * GB, GB)])
        plsc.subcore_barrier()

        pltpu.sync_copy(idx_hbm.at[pl.ds(s * SNB, SNB)], idx_v)

        def body(j2, carry):
            j0 = 2 * j2
            j1 = j0 + 1

            @pl.when(j2 > 0)
            def _():
                pltpu.make_async_copy(mb0, acc.at[pl.ds(0, GB)], ss0).wait()

            l0 = pltpu.async_copy(
                msg_hbm.at[c, pl.ds((s * SNB + j0) * GB, GB)], mb0, ls0)

            @pl.when(j2 > 0)
            def _():
                pltpu.make_async_copy(mb1, acc.at[pl.ds(0, GB)], ss1).wait()

            l1 = pltpu.async_copy(
                msg_hbm.at[c, pl.ds((s * SNB + j1) * GB, GB)], mb1, ls1)
            l0.wait()
            pltpu.async_copy(mb0, acc.at[idx_v.at[j0]], ss0, add=True)
            l1.wait()
            pltpu.async_copy(mb1, acc.at[idx_v.at[j1]], ss1, add=True)
            return carry

        lax.fori_loop(0, SNB // 2, body, 0)
        pltpu.make_async_copy(mb0, acc.at[pl.ds(0, GB)], ss0).wait()
        pltpu.make_async_copy(mb1, acc.at[pl.ds(0, GB)], ss1).wait()
        plsc.subcore_barrier()
        pltpu.sync_copy(acc.at[pl.ds(s * NPT, NPT)],
                        out_hbm.at[pl.ds(s * NPT, NPT), pl.ds(c * FH, FH)])

    return k(msg, idx2d)


BE = 2048
BN = 1024


def _rbf_kernel(dist_c):
    coeff = -0.5 / (CUT / NG) ** 2
    step = CUT / (NG - 1)

    def body(d_ref, rbf_ref, cut_ref):
        d = d_ref[...]
        offset = lax.broadcasted_iota(
            jnp.int32, (1, NG), 1).astype(jnp.float32) * step
        diff = d - offset
        rbf_ref[...] = jnp.exp(coeff * diff * diff)
        cut_ref[...] = 0.5 * (jnp.cos(d * (np.pi / CUT)) + 1.0) * (
            d < CUT).astype(jnp.float32)

    return pl.pallas_call(
        body,
        grid=(E_PAD // BE,),
        in_specs=[pl.BlockSpec((BE, 1), lambda i: (i, 0))],
        out_specs=[pl.BlockSpec((BE, NG), lambda i: (i, 0)),
                   pl.BlockSpec((BE, 1), lambda i: (i, 0))],
        out_shape=[jax.ShapeDtypeStruct((E_PAD, NG), jnp.float32),
                   jax.ShapeDtypeStruct((E_PAD, 1), jnp.float32)],
    )(dist_c)


def _emb_kernel(z_c, emb_pad, w1t, b1):
    def body(z_ref, emb_ref, w_ref, b_ref, h_ref, hA_ref):
        onehot = (z_ref[...] == lax.broadcasted_iota(jnp.int32, (1, 128), 1)
                  ).astype(jnp.float32)
        h = jnp.dot(onehot, emb_ref[...], preferred_element_type=jnp.float32)
        h_ref[...] = h
        hA = jnp.dot(h, w_ref[...],
                     preferred_element_type=jnp.float32) + b_ref[...]
        hA_ref[...] = _pack_bf16_pair(hA)

    return pl.pallas_call(
        body,
        grid=(N_PAD // BN,),
        in_specs=[pl.BlockSpec((BN, 1), lambda i: (i, 0)),
                  pl.BlockSpec((128, HID), lambda i: (0, 0)),
                  pl.BlockSpec((HID, HID), lambda i: (0, 0)),
                  pl.BlockSpec((1, HID), lambda i: (0, 0))],
        out_specs=[pl.BlockSpec((BN, HID), lambda i: (i, 0)),
                   pl.BlockSpec((BN, HID // 2), lambda i: (i, 0))],
        out_shape=[jax.ShapeDtypeStruct((N_PAD, HID), jnp.float32),
                   jax.ShapeDtypeStruct((N_PAD, HID // 2), jnp.int32)],
    )(z_c, emb_pad, w1t, b1)


def _filter_kernel(rbf, cut, g, w1t, b1, w2t, b2):
    def body(rbf_ref, cut_ref, g_ref, w1_ref, b1_ref, w2_ref, b2_ref, o_ref):
        t = jnp.dot(rbf_ref[...], w1_ref[...],
                    preferred_element_type=jnp.float32) + b1_ref[...]
        t = _ssp(t)
        w = jnp.dot(t, w2_ref[...],
                    preferred_element_type=jnp.float32) + b2_ref[...]
        g_lo, g_hi = _unpack_bf16_pair(g_ref[...])
        cw = cut_ref[...]
        o_ref[0] = g_lo * w[:, :FH] * cw
        o_ref[1] = g_hi * w[:, FH:] * cw

    return pl.pallas_call(
        body,
        grid=(E_PAD // BE,),
        in_specs=[pl.BlockSpec((BE, NG), lambda i: (i, 0)),
                  pl.BlockSpec((BE, 1), lambda i: (i, 0)),
                  pl.BlockSpec((BE, HID // 2), lambda i: (i, 0)),
                  pl.BlockSpec((NG, HID), lambda i: (0, 0)),
                  pl.BlockSpec((1, HID), lambda i: (0, 0)),
                  pl.BlockSpec((HID, HID), lambda i: (0, 0)),
                  pl.BlockSpec((1, HID), lambda i: (0, 0))],
        out_specs=pl.BlockSpec((NC, BE, FH), lambda i: (0, i, 0)),
        out_shape=jax.ShapeDtypeStruct((NC, E_PAD, FH), jnp.float32),
    )(rbf, cut, g, w1t, b1, w2t, b2)


def _node_kernel(agg, h, w2t, b2, w3t, b3, wnt=None, bn=None):
    dual = wnt is not None

    def body(agg_ref, h_ref, w2_ref, b2_ref, w3_ref, b3_ref, *rest):
        if dual:
            wn_ref, bn_ref, hn_ref, hA_ref = rest
        else:
            (hn_ref,) = rest
        t = jnp.dot(agg_ref[...], w2_ref[...],
                    preferred_element_type=jnp.float32) + b2_ref[...]
        t = _ssp(t)
        t = jnp.dot(t, w3_ref[...],
                    preferred_element_type=jnp.float32) + b3_ref[...]
        hn = h_ref[...] + t
        hn_ref[...] = hn
        if dual:
            hA = jnp.dot(hn, wn_ref[...],
                         preferred_element_type=jnp.float32) + bn_ref[...]
            hA_ref[...] = _pack_bf16_pair(hA)

    full = lambda i: (0, 0)
    blk = lambda i: (i, 0)
    in_specs = [pl.BlockSpec((BN, HID), blk), pl.BlockSpec((BN, HID), blk),
                pl.BlockSpec((HID, HID), full), pl.BlockSpec((1, HID), full),
                pl.BlockSpec((HID, HID), full), pl.BlockSpec((1, HID), full)]
    args = [agg, h, w2t, b2, w3t, b3]
    if dual:
        in_specs += [pl.BlockSpec((HID, HID), full), pl.BlockSpec((1, HID), full)]
        args += [wnt, bn]
        out_specs = [pl.BlockSpec((BN, HID), blk),
                     pl.BlockSpec((BN, HID // 2), blk)]
        out_shape = [jax.ShapeDtypeStruct((N_PAD, HID), jnp.float32),
                     jax.ShapeDtypeStruct((N_PAD, HID // 2), jnp.int32)]
    else:
        out_specs = pl.BlockSpec((BN, HID), blk)
        out_shape = jax.ShapeDtypeStruct((N_PAD, HID), jnp.float32)

    return pl.pallas_call(
        body,
        grid=(N_PAD // BN,),
        in_specs=in_specs,
        out_specs=out_specs,
        out_shape=out_shape,
    )(*args)


def _readout_kernel(h, batch_c, r1wt, r1b, r2w, r2b):
    def body(h_ref, b_ref, w1_ref, b1_ref, w2_ref, b2_ref, o_ref):
        t = jnp.dot(h_ref[...], w1_ref[...],
                    preferred_element_type=jnp.float32) + b1_ref[...]
        t = _ssp(t)
        e = jnp.sum(t * w2_ref[...], axis=1, keepdims=True) + b2_ref[...]
        onehot = (b_ref[...] == lax.broadcasted_iota(jnp.int32, (1, NMOL), 1)
                  ).astype(jnp.float32)
        mol = jnp.sum(onehot * e, axis=0, keepdims=True)

        @pl.when(pl.program_id(0) == 0)
        def _():
            o_ref[...] = jnp.zeros_like(o_ref)

        o_ref[...] += mol

    return pl.pallas_call(
        body,
        grid=(N_PAD // BN,),
        in_specs=[pl.BlockSpec((BN, HID), lambda i: (i, 0)),
                  pl.BlockSpec((BN, 1), lambda i: (i, 0)),
                  pl.BlockSpec((HID, HID // 2), lambda i: (0, 0)),
                  pl.BlockSpec((1, HID // 2), lambda i: (0, 0)),
                  pl.BlockSpec((1, HID // 2), lambda i: (0, 0)),
                  pl.BlockSpec((1, 1), lambda i: (0, 0))],
        out_specs=pl.BlockSpec((1, NMOL), lambda i: (0, 0)),
        out_shape=jax.ShapeDtypeStruct((1, NMOL), jnp.float32),
    )(h, batch_c, r1wt, r1b, r2w, r2b)


def kernel(z, edge_index, edge_attr, batch, emb, fw1, fb1, fw2, fb2,
           a1w, a1b, a2w, a2b, a3w, a3b, r1w, r1b, r2w, r2b):
    row = edge_index[0]
    col = edge_index[1]
    dist = edge_attr[:, 0]

    pad = E_PAD - N_EDGES
    col2d = jnp.concatenate(
        [col, jnp.zeros((pad,), col.dtype)]).reshape(E_PAD // GB, GB)
    row2d = jnp.concatenate(
        [row, jnp.zeros((pad,), row.dtype)]).reshape(E_PAD // GB, GB)
    dist_c = jnp.concatenate(
        [dist, jnp.full((pad,), 2.0 * CUT, dist.dtype)]).reshape(E_PAD, 1)

    emb_pad = jnp.zeros((128, HID), jnp.float32).at[: emb.shape[0]].set(emb)
    npad = N_PAD - N_NODES
    z_c = jnp.concatenate(
        [z.astype(jnp.int32), jnp.zeros((npad,), jnp.int32)]).reshape(N_PAD, 1)
    batch_c = jnp.concatenate(
        [batch.astype(jnp.int32),
         jnp.full((npad,), NMOL, jnp.int32)]).reshape(N_PAD, 1)

    fw1t = jnp.swapaxes(fw1, 1, 2)
    fw2t = jnp.swapaxes(fw2, 1, 2)
    a1wt = jnp.swapaxes(a1w, 1, 2)
    a2wt = jnp.swapaxes(a2w, 1, 2)
    a3wt = jnp.swapaxes(a3w, 1, 2)
    fb1_2 = fb1[:, None, :]
    fb2_2 = fb2[:, None, :]
    a1b_2 = a1b[:, None, :]
    a2b_2 = a2b[:, None, :]
    a3b_2 = a3b[:, None, :]
    r1wt = r1w.T
    r1b_2 = r1b[None, :]
    r2b_2 = r2b[None, :]

    rbf, cut = _rbf_kernel(dist_c)
    h, hA = _emb_kernel(z_c, emb_pad, a1wt[0], a1b_2[0])

    for l in range(NL):
        g = _sc_gather(hA, col2d)
        msg = _filter_kernel(rbf, cut, g, fw1t[l], fb1_2[l], fw2t[l], fb2_2[l])
        agg = _sc_scatter_add(msg, row2d)
        if l < NL - 1:
            h, hA = _node_kernel(agg, h, a2wt[l], a2b_2[l], a3wt[l], a3b_2[l],
                                 a1wt[l + 1], a1b_2[l + 1])
        else:
            h = _node_kernel(agg, h, a2wt[l], a2b_2[l], a3wt[l], a3b_2[l])

    mol = _readout_kernel(h, batch_c, r1wt, r1b_2, r2w, r2b_2)
    return mol[0]


# speedup vs baseline: 2.8343x; 1.3168x over previous
import functools

import jax
import jax.numpy as jnp
import numpy as np
from jax import lax
from jax.experimental import pallas as pl
from jax.experimental.pallas import tpu as pltpu
from jax.experimental.pallas import tpu_sc as plsc

HID = 256
NG = 64
NL = 4
CUT = 5.0
NMOL = 64
N_NODES = 10000
N_PAD = 10240
N_EDGES = 160000
LOG2 = float(np.log(2.0))

NC = 2
NS = 16
NW = NC * NS

GB = 128
E_PAD = 163840
GNB = E_PAD // (NW * GB)
SNB = E_PAD // (NS * GB)
NPT = N_PAD // NS
FH = HID // NC


def _ssp(x):
    return jnp.maximum(x, 0.0) + jnp.log1p(jnp.exp(-jnp.abs(x))) - LOG2


def _pack_bf16_pair(x):
    lo = lax.bitcast_convert_type(
        x[:, :FH].astype(jnp.bfloat16), jnp.uint16).astype(jnp.int32)
    hi = lax.bitcast_convert_type(
        x[:, FH:].astype(jnp.bfloat16), jnp.uint16).astype(jnp.int32)
    return (hi << 16) | lo


def _unpack_bf16_pair(p):
    lo = lax.bitcast_convert_type(p << 16, jnp.float32)
    hi = lax.bitcast_convert_type(
        p & jnp.int32(np.uint32(0xFFFF0000)), jnp.float32)
    return lo, hi


def _sc_gather(table, idx2d):
    mesh = plsc.VectorSubcoreMesh(core_axis_name="c", subcore_axis_name="s")

    @functools.partial(
        pl.kernel,
        mesh=mesh,
        out_type=jax.ShapeDtypeStruct((E_PAD, HID // 2), jnp.int32),
        scratch_types=[
            pltpu.VMEM((GNB, GB), jnp.int32),
            pltpu.VMEM((GB, HID // 2), jnp.int32),
            pltpu.VMEM((GB, HID // 2), jnp.int32),
            pltpu.SemaphoreType.DMA,
            pltpu.SemaphoreType.DMA,
            pltpu.SemaphoreType.DMA,
            pltpu.SemaphoreType.DMA,
        ],
    )
    def k(table_hbm, idx_hbm, out_hbm, idx_v, buf0, buf1, gs0, gs1, ws0, ws1):
        wid = lax.axis_index("s") * NC + lax.axis_index("c")
        blk0 = wid * GNB
        pltpu.sync_copy(idx_hbm.at[pl.ds(blk0, GNB)], idx_v)

        def body(j2, carry):
            j0 = 2 * j2
            j1 = j0 + 1

            @pl.when(j2 > 0)
            def _():
                pltpu.make_async_copy(
                    buf0, out_hbm.at[pl.ds(0, GB)], ws0).wait()

            g0 = pltpu.async_copy(table_hbm.at[idx_v.at[j0]], buf0, gs0)

            @pl.when(j2 > 0)
            def _():
                pltpu.make_async_copy(
                    buf1, out_hbm.at[pl.ds(0, GB)], ws1).wait()

            g1 = pltpu.async_copy(table_hbm.at[idx_v.at[j1]], buf1, gs1)
            g0.wait()
            pltpu.async_copy(buf0, out_hbm.at[pl.ds((blk0 + j0) * GB, GB)], ws0)
            g1.wait()
            pltpu.async_copy(buf1, out_hbm.at[pl.ds((blk0 + j1) * GB, GB)], ws1)
            return carry

        lax.fori_loop(0, GNB // 2, body, 0)
        pltpu.make_async_copy(buf0, out_hbm.at[pl.ds(0, GB)], ws0).wait()
        pltpu.make_async_copy(buf1, out_hbm.at[pl.ds(0, GB)], ws1).wait()

    return k(table, idx2d)


def _sc_scatter_add(msg, idx2d):
    mesh = plsc.VectorSubcoreMesh(core_axis_name="c", subcore_axis_name="s")

    @functools.partial(
        pl.kernel,
        mesh=mesh,
        out_type=jax.ShapeDtypeStruct((N_PAD, HID), jnp.float32),
        scratch_types=[
            pltpu.VMEM((SNB, GB), jnp.int32),
            pltpu.VMEM((GB, FH), jnp.float32),
            pltpu.VMEM((GB, FH), jnp.float32),
            pltpu.VMEM_SHARED((N_PAD, FH), jnp.float32),
            pltpu.SemaphoreType.DMA,
            pltpu.SemaphoreType.DMA,
            pltpu.SemaphoreType.DMA,
            pltpu.SemaphoreType.DMA,
        ],
    )
    def k(msg_hbm, idx_hbm, out_hbm, idx_v, mb0, mb1, acc, ls0, ls1, ss0, ss1):
        c = lax.axis_index("c")
        s = lax.axis_index("s")

        def zrow(i, carry):
            for kk in range(FH // 16):
                mb0[i, pl.ds(kk * 16, 16)] = jnp.zeros((16,), jnp.float32)
            return carry

        lax.fori_loop(0, GB, zrow, 0)
        for t in range(NPT // GB):
            pltpu.sync_copy(mb0, acc.at[pl.ds(s * NPT + t * GB, GB)])
        plsc.subcore_barrier()

        pltpu.sync_copy(idx_hbm.at[pl.ds(s * SNB, SNB)], idx_v)

        def body(j2, carry):
            j0 = 2 * j2
            j1 = j0 + 1

            @pl.when(j2 > 0)
            def _():
                pltpu.make_async_copy(mb0, acc.at[pl.ds(0, GB)], ss0).wait()

            l0 = pltpu.async_copy(
                msg_hbm.at[c, pl.ds((s * SNB + j0) * GB, GB)], mb0, ls0)

            @pl.when(j2 > 0)
            def _():
                pltpu.make_async_copy(mb1, acc.at[pl.ds(0, GB)], ss1).wait()

            l1 = pltpu.async_copy(
                msg_hbm.at[c, pl.ds((s * SNB + j1) * GB, GB)], mb1, ls1)
            l0.wait()
            pltpu.async_copy(mb0, acc.at[idx_v.at[j0]], ss0, add=True)
            l1.wait()
            pltpu.async_copy(mb1, acc.at[idx_v.at[j1]], ss1, add=True)
            return carry

        lax.fori_loop(0, SNB // 2, body, 0)
        pltpu.make_async_copy(mb0, acc.at[pl.ds(0, GB)], ss0).wait()
        pltpu.make_async_copy(mb1, acc.at[pl.ds(0, GB)], ss1).wait()
        plsc.subcore_barrier()
        pltpu.sync_copy(acc.at[pl.ds(s * NPT, NPT)],
                        out_hbm.at[pl.ds(s * NPT, NPT), pl.ds(c * FH, FH)])

    return k(msg, idx2d)


BE = 2048
BN = 1024


def _rbf_kernel(dist_c):
    coeff = -0.5 / (CUT / NG) ** 2
    step = CUT / (NG - 1)

    def body(d_ref, rbf_ref, cut_ref):
        d = d_ref[...]
        offset = lax.broadcasted_iota(
            jnp.int32, (1, NG), 1).astype(jnp.float32) * step
        diff = d - offset
        rbf_ref[...] = jnp.exp(coeff * diff * diff)
        cut_ref[...] = 0.5 * (jnp.cos(d * (np.pi / CUT)) + 1.0) * (
            d < CUT).astype(jnp.float32)

    return pl.pallas_call(
        body,
        grid=(E_PAD // BE,),
        in_specs=[pl.BlockSpec((BE, 1), lambda i: (i, 0))],
        out_specs=[pl.BlockSpec((BE, NG), lambda i: (i, 0)),
                   pl.BlockSpec((BE, 1), lambda i: (i, 0))],
        out_shape=[jax.ShapeDtypeStruct((E_PAD, NG), jnp.float32),
                   jax.ShapeDtypeStruct((E_PAD, 1), jnp.float32)],
    )(dist_c)


def _emb_kernel(z_c, emb_pad, w1t, b1):
    def body(z_ref, emb_ref, w_ref, b_ref, h_ref, hA_ref):
        onehot = (z_ref[...] == lax.broadcasted_iota(jnp.int32, (1, 128), 1)
                  ).astype(jnp.float32)
        h = jnp.dot(onehot, emb_ref[...], preferred_element_type=jnp.float32)
        h_ref[...] = h
        hA = jnp.dot(h, w_ref[...],
                     preferred_element_type=jnp.float32) + b_ref[...]
        hA_ref[...] = _pack_bf16_pair(hA)

    return pl.pallas_call(
        body,
        grid=(N_PAD // BN,),
        in_specs=[pl.BlockSpec((BN, 1), lambda i: (i, 0)),
                  pl.BlockSpec((128, HID), lambda i: (0, 0)),
                  pl.BlockSpec((HID, HID), lambda i: (0, 0)),
                  pl.BlockSpec((1, HID), lambda i: (0, 0))],
        out_specs=[pl.BlockSpec((BN, HID), lambda i: (i, 0)),
                   pl.BlockSpec((BN, HID // 2), lambda i: (i, 0))],
        out_shape=[jax.ShapeDtypeStruct((N_PAD, HID), jnp.float32),
                   jax.ShapeDtypeStruct((N_PAD, HID // 2), jnp.int32)],
    )(z_c, emb_pad, w1t, b1)


def _filter_kernel(rbf, cut, g, w1t, b1, w2t, b2):
    def body(rbf_ref, cut_ref, g_ref, w1_ref, b1_ref, w2_ref, b2_ref, o_ref):
        t = jnp.dot(rbf_ref[...].astype(jnp.bfloat16), w1_ref[...],
                    preferred_element_type=jnp.float32) + b1_ref[...]
        t = _ssp(t)
        w = jnp.dot(t.astype(jnp.bfloat16), w2_ref[...],
                    preferred_element_type=jnp.float32) + b2_ref[...]
        g_lo, g_hi = _unpack_bf16_pair(g_ref[...])
        cw = cut_ref[...]
        o_ref[0] = g_lo * w[:, :FH] * cw
        o_ref[1] = g_hi * w[:, FH:] * cw

    return pl.pallas_call(
        body,
        grid=(E_PAD // BE,),
        in_specs=[pl.BlockSpec((BE, NG), lambda i: (i, 0)),
                  pl.BlockSpec((BE, 1), lambda i: (i, 0)),
                  pl.BlockSpec((BE, HID // 2), lambda i: (i, 0)),
                  pl.BlockSpec((NG, HID), lambda i: (0, 0)),
                  pl.BlockSpec((1, HID), lambda i: (0, 0)),
                  pl.BlockSpec((HID, HID), lambda i: (0, 0)),
                  pl.BlockSpec((1, HID), lambda i: (0, 0))],
        out_specs=pl.BlockSpec((NC, BE, FH), lambda i: (0, i, 0)),
        out_shape=jax.ShapeDtypeStruct((NC, E_PAD, FH), jnp.float32),
    )(rbf, cut, g, w1t, b1, w2t, b2)


def _node_kernel(agg, h, w2t, b2, w3t, b3, wnt=None, bn=None):
    dual = wnt is not None

    def body(agg_ref, h_ref, w2_ref, b2_ref, w3_ref, b3_ref, *rest):
        if dual:
            wn_ref, bn_ref, hn_ref, hA_ref = rest
        else:
            (hn_ref,) = rest
        t = jnp.dot(agg_ref[...], w2_ref[...],
                    preferred_element_type=jnp.float32) + b2_ref[...]
        t = _ssp(t)
        t = jnp.dot(t, w3_ref[...],
                    preferred_element_type=jnp.float32) + b3_ref[...]
        hn = h_ref[...] + t
        hn_ref[...] = hn
        if dual:
            hA = jnp.dot(hn, wn_ref[...],
                         preferred_element_type=jnp.float32) + bn_ref[...]
            hA_ref[...] = _pack_bf16_pair(hA)

    full = lambda i: (0, 0)
    blk = lambda i: (i, 0)
    in_specs = [pl.BlockSpec((BN, HID), blk), pl.BlockSpec((BN, HID), blk),
                pl.BlockSpec((HID, HID), full), pl.BlockSpec((1, HID), full),
                pl.BlockSpec((HID, HID), full), pl.BlockSpec((1, HID), full)]
    args = [agg, h, w2t, b2, w3t, b3]
    if dual:
        in_specs += [pl.BlockSpec((HID, HID), full), pl.BlockSpec((1, HID), full)]
        args += [wnt, bn]
        out_specs = [pl.BlockSpec((BN, HID), blk),
                     pl.BlockSpec((BN, HID // 2), blk)]
        out_shape = [jax.ShapeDtypeStruct((N_PAD, HID), jnp.float32),
                     jax.ShapeDtypeStruct((N_PAD, HID // 2), jnp.int32)]
    else:
        out_specs = pl.BlockSpec((BN, HID), blk)
        out_shape = jax.ShapeDtypeStruct((N_PAD, HID), jnp.float32)

    return pl.pallas_call(
        body,
        grid=(N_PAD // BN,),
        in_specs=in_specs,
        out_specs=out_specs,
        out_shape=out_shape,
    )(*args)


def _readout_kernel(h, batch_c, r1wt, r1b, r2w, r2b):
    def body(h_ref, b_ref, w1_ref, b1_ref, w2_ref, b2_ref, o_ref):
        t = jnp.dot(h_ref[...], w1_ref[...],
                    preferred_element_type=jnp.float32) + b1_ref[...]
        t = _ssp(t)
        e = jnp.sum(t * w2_ref[...], axis=1, keepdims=True) + b2_ref[...]
        onehot = (b_ref[...] == lax.broadcasted_iota(jnp.int32, (1, NMOL), 1)
                  ).astype(jnp.float32)
        mol = jnp.sum(onehot * e, axis=0, keepdims=True)

        @pl.when(pl.program_id(0) == 0)
        def _():
            o_ref[...] = jnp.zeros_like(o_ref)

        o_ref[...] += mol

    return pl.pallas_call(
        body,
        grid=(N_PAD // BN,),
        in_specs=[pl.BlockSpec((BN, HID), lambda i: (i, 0)),
                  pl.BlockSpec((BN, 1), lambda i: (i, 0)),
                  pl.BlockSpec((HID, HID // 2), lambda i: (0, 0)),
                  pl.BlockSpec((1, HID // 2), lambda i: (0, 0)),
                  pl.BlockSpec((1, HID // 2), lambda i: (0, 0)),
                  pl.BlockSpec((1, 1), lambda i: (0, 0))],
        out_specs=pl.BlockSpec((1, NMOL), lambda i: (0, 0)),
        out_shape=jax.ShapeDtypeStruct((1, NMOL), jnp.float32),
    )(h, batch_c, r1wt, r1b, r2w, r2b)


def kernel(z, edge_index, edge_attr, batch, emb, fw1, fb1, fw2, fb2,
           a1w, a1b, a2w, a2b, a3w, a3b, r1w, r1b, r2w, r2b):
    row = edge_index[0]
    col = edge_index[1]
    dist = edge_attr[:, 0]

    pad = E_PAD - N_EDGES
    spread = jnp.arange(pad, dtype=jnp.int32) % N_NODES
    col2d = jnp.concatenate([col, spread]).reshape(E_PAD // GB, GB)
    row2d = jnp.concatenate([row, spread]).reshape(E_PAD // GB, GB)
    dist_c = jnp.concatenate(
        [dist, jnp.full((pad,), 2.0 * CUT, dist.dtype)]).reshape(E_PAD, 1)

    emb_pad = jnp.zeros((128, HID), jnp.float32).at[: emb.shape[0]].set(emb)
    npad = N_PAD - N_NODES
    z_c = jnp.concatenate(
        [z.astype(jnp.int32), jnp.zeros((npad,), jnp.int32)]).reshape(N_PAD, 1)
    batch_c = jnp.concatenate(
        [batch.astype(jnp.int32),
         jnp.full((npad,), NMOL, jnp.int32)]).reshape(N_PAD, 1)

    fw1t = jnp.swapaxes(fw1, 1, 2).astype(jnp.bfloat16)
    fw2t = jnp.swapaxes(fw2, 1, 2).astype(jnp.bfloat16)
    a1wt = jnp.swapaxes(a1w, 1, 2)
    a2wt = jnp.swapaxes(a2w, 1, 2)
    a3wt = jnp.swapaxes(a3w, 1, 2)
    fb1_2 = fb1[:, None, :]
    fb2_2 = fb2[:, None, :]
    a1b_2 = a1b[:, None, :]
    a2b_2 = a2b[:, None, :]
    a3b_2 = a3b[:, None, :]
    r1wt = r1w.T
    r1b_2 = r1b[None, :]
    r2b_2 = r2b[None, :]

    rbf, cut = _rbf_kernel(dist_c)
    h, hA = _emb_kernel(z_c, emb_pad, a1wt[0], a1b_2[0])

    for l in range(NL):
        g = _sc_gather(hA, col2d)
        msg = _filter_kernel(rbf, cut, g, fw1t[l], fb1_2[l], fw2t[l], fb2_2[l])
        agg = _sc_scatter_add(msg, row2d)
        if l < NL - 1:
            h, hA = _node_kernel(agg, h, a2wt[l], a2b_2[l], a3wt[l], a3b_2[l],
                                 a1wt[l + 1], a1b_2[l + 1])
        else:
            h = _node_kernel(agg, h, a2wt[l], a2b_2[l], a3wt[l], a3b_2[l])

    mol = _readout_kernel(h, batch_c, r1wt, r1b_2, r2w, r2b_2)
    return mol[0]


# speedup vs baseline: 2.9267x; 1.0326x over previous
import functools

import jax
import jax.numpy as jnp
import numpy as np
from jax import lax
from jax.experimental import pallas as pl
from jax.experimental.pallas import tpu as pltpu
from jax.experimental.pallas import tpu_sc as plsc

HID = 256
NG = 64
NL = 4
CUT = 5.0
NMOL = 64
N_NODES = 10000
N_PAD = 10240
N_EDGES = 160000
LOG2 = float(np.log(2.0))

NC = 2
NS = 16
NW = NC * NS

GB = 128
E_PAD = 163840
GNB = E_PAD // (NW * GB)
SNB = E_PAD // (NS * GB)
NPT = N_PAD // NS
FH = HID // NC


def _ssp(x):
    return jnp.maximum(x, 0.0) + jnp.log1p(jnp.exp(-jnp.abs(x))) - LOG2


def _pack_bf16_pair(x):
    lo = lax.bitcast_convert_type(
        x[:, :FH].astype(jnp.bfloat16), jnp.uint16).astype(jnp.int32)
    hi = lax.bitcast_convert_type(
        x[:, FH:].astype(jnp.bfloat16), jnp.uint16).astype(jnp.int32)
    return (hi << 16) | lo


def _unpack_bf16_pair(p):
    lo = lax.bitcast_convert_type(p << 16, jnp.float32)
    hi = lax.bitcast_convert_type(
        p & jnp.int32(np.uint32(0xFFFF0000)), jnp.float32)
    return lo, hi


def _sc_gather(table, idx2d):
    mesh = plsc.VectorSubcoreMesh(core_axis_name="c", subcore_axis_name="s")

    @functools.partial(
        pl.kernel,
        mesh=mesh,
        out_type=jax.ShapeDtypeStruct((E_PAD, HID // 2), jnp.int32),
        scratch_types=[
            pltpu.VMEM((GNB, GB), jnp.int32),
            pltpu.VMEM((GB, HID // 2), jnp.int32),
            pltpu.VMEM((GB, HID // 2), jnp.int32),
            pltpu.VMEM_SHARED((N_PAD, HID // 2), jnp.int32),
            pltpu.SemaphoreType.DMA,
            pltpu.SemaphoreType.DMA,
            pltpu.SemaphoreType.DMA,
            pltpu.SemaphoreType.DMA,
        ],
    )
    def k(table_hbm, idx_hbm, out_hbm, idx_v, buf0, buf1, tbl,
          gs0, gs1, ws0, ws1):
        wid = lax.axis_index("s") * NC + lax.axis_index("c")
        s = lax.axis_index("s")
        blk0 = wid * GNB
        pltpu.sync_copy(table_hbm.at[pl.ds(s * NPT, NPT)],
                        tbl.at[pl.ds(s * NPT, NPT)])
        pltpu.sync_copy(idx_hbm.at[pl.ds(blk0, GNB)], idx_v)
        plsc.subcore_barrier()

        def body(j2, carry):
            j0 = 2 * j2
            j1 = j0 + 1

            @pl.when(j2 > 0)
            def _():
                pltpu.make_async_copy(
                    buf0, out_hbm.at[pl.ds(0, GB)], ws0).wait()

            g0 = pltpu.async_copy(tbl.at[idx_v.at[j0]], buf0, gs0)

            @pl.when(j2 > 0)
            def _():
                pltpu.make_async_copy(
                    buf1, out_hbm.at[pl.ds(0, GB)], ws1).wait()

            g1 = pltpu.async_copy(tbl.at[idx_v.at[j1]], buf1, gs1)
            g0.wait()
            pltpu.async_copy(buf0, out_hbm.at[pl.ds((blk0 + j0) * GB, GB)], ws0)
            g1.wait()
            pltpu.async_copy(buf1, out_hbm.at[pl.ds((blk0 + j1) * GB, GB)], ws1)
            return carry

        lax.fori_loop(0, GNB // 2, body, 0)
        pltpu.make_async_copy(buf0, out_hbm.at[pl.ds(0, GB)], ws0).wait()
        pltpu.make_async_copy(buf1, out_hbm.at[pl.ds(0, GB)], ws1).wait()

    return k(table, idx2d)


def _sc_scatter_add(msg, idx2d):
    mesh = plsc.VectorSubcoreMesh(core_axis_name="c", subcore_axis_name="s")

    @functools.partial(
        pl.kernel,
        mesh=mesh,
        out_type=jax.ShapeDtypeStruct((N_PAD, HID), jnp.float32),
        scratch_types=[
            pltpu.VMEM((SNB, GB), jnp.int32),
            pltpu.VMEM((GB, FH), jnp.float32),
            pltpu.VMEM((GB, FH), jnp.float32),
            pltpu.VMEM_SHARED((N_PAD, FH), jnp.float32),
            pltpu.SemaphoreType.DMA,
            pltpu.SemaphoreType.DMA,
            pltpu.SemaphoreType.DMA,
            pltpu.SemaphoreType.DMA,
        ],
    )
    def k(msg_hbm, idx_hbm, out_hbm, idx_v, mb0, mb1, acc, ls0, ls1, ss0, ss1):
        c = lax.axis_index("c")
        s = lax.axis_index("s")

        def zrow(i, carry):
            for kk in range(FH // 16):
                mb0[i, pl.ds(kk * 16, 16)] = jnp.zeros((16,), jnp.float32)
            return carry

        lax.fori_loop(0, GB, zrow, 0)
        for t in range(NPT // GB):
            pltpu.sync_copy(mb0, acc.at[pl.ds(s * NPT + t * GB, GB)])
        plsc.subcore_barrier()

        pltpu.sync_copy(idx_hbm.at[pl.ds(s * SNB, SNB)], idx_v)

        def body(j2, carry):
            j0 = 2 * j2
            j1 = j0 + 1

            @pl.when(j2 > 0)
            def _():
                pltpu.make_async_copy(mb0, acc.at[pl.ds(0, GB)], ss0).wait()

            l0 = pltpu.async_copy(
                msg_hbm.at[c, pl.ds((s * SNB + j0) * GB, GB)], mb0, ls0)

            @pl.when(j2 > 0)
            def _():
                pltpu.make_async_copy(mb1, acc.at[pl.ds(0, GB)], ss1).wait()

            l1 = pltpu.async_copy(
                msg_hbm.at[c, pl.ds((s * SNB + j1) * GB, GB)], mb1, ls1)
            l0.wait()
            pltpu.async_copy(mb0, acc.at[idx_v.at[j0]], ss0, add=True)
            l1.wait()
            pltpu.async_copy(mb1, acc.at[idx_v.at[j1]], ss1, add=True)
            return carry

        lax.fori_loop(0, SNB // 2, body, 0)
        pltpu.make_async_copy(mb0, acc.at[pl.ds(0, GB)], ss0).wait()
        pltpu.make_async_copy(mb1, acc.at[pl.ds(0, GB)], ss1).wait()
        plsc.subcore_barrier()
        pltpu.sync_copy(acc.at[pl.ds(s * NPT, NPT)],
                        out_hbm.at[pl.ds(s * NPT, NPT), pl.ds(c * FH, FH)])

    return k(msg, idx2d)


BE = 2048
BN = 1024


def _rbf_kernel(dist_c):
    coeff = -0.5 / (CUT / NG) ** 2
    step = CUT / (NG - 1)

    def body(d_ref, rbf_ref, cut_ref):
        d = d_ref[...]
        offset = lax.broadcasted_iota(
            jnp.int32, (1, NG), 1).astype(jnp.float32) * step
        diff = d - offset
        rbf_ref[...] = jnp.exp(coeff * diff * diff)
        cut_ref[...] = 0.5 * (jnp.cos(d * (np.pi / CUT)) + 1.0) * (
            d < CUT).astype(jnp.float32)

    return pl.pallas_call(
        body,
        grid=(E_PAD // BE,),
        in_specs=[pl.BlockSpec((BE, 1), lambda i: (i, 0))],
        out_specs=[pl.BlockSpec((BE, NG), lambda i: (i, 0)),
                   pl.BlockSpec((BE, 1), lambda i: (i, 0))],
        out_shape=[jax.ShapeDtypeStruct((E_PAD, NG), jnp.float32),
                   jax.ShapeDtypeStruct((E_PAD, 1), jnp.float32)],
    )(dist_c)


def _emb_kernel(z_c, emb_pad, w1t, b1):
    def body(z_ref, emb_ref, w_ref, b_ref, h_ref, hA_ref):
        onehot = (z_ref[...] == lax.broadcasted_iota(jnp.int32, (1, 128), 1)
                  ).astype(jnp.float32)
        h = jnp.dot(onehot, emb_ref[...], preferred_element_type=jnp.float32)
        h_ref[...] = h
        hA = jnp.dot(h.astype(jnp.bfloat16), w_ref[...],
                     preferred_element_type=jnp.float32) + b_ref[...]
        hA_ref[...] = _pack_bf16_pair(hA)

    return pl.pallas_call(
        body,
        grid=(N_PAD // BN,),
        in_specs=[pl.BlockSpec((BN, 1), lambda i: (i, 0)),
                  pl.BlockSpec((128, HID), lambda i: (0, 0)),
                  pl.BlockSpec((HID, HID), lambda i: (0, 0)),
                  pl.BlockSpec((1, HID), lambda i: (0, 0))],
        out_specs=[pl.BlockSpec((BN, HID), lambda i: (i, 0)),
                   pl.BlockSpec((BN, HID // 2), lambda i: (i, 0))],
        out_shape=[jax.ShapeDtypeStruct((N_PAD, HID), jnp.float32),
                   jax.ShapeDtypeStruct((N_PAD, HID // 2), jnp.int32)],
    )(z_c, emb_pad, w1t, b1)


def _filter_kernel(rbf, cut, g, w1t, b1, w2t, b2):
    def body(rbf_ref, cut_ref, g_ref, w1_ref, b1_ref, w2_ref, b2_ref, o_ref):
        t = jnp.dot(rbf_ref[...].astype(jnp.bfloat16), w1_ref[...],
                    preferred_element_type=jnp.float32) + b1_ref[...]
        t = _ssp(t)
        w = jnp.dot(t.astype(jnp.bfloat16), w2_ref[...],
                    preferred_element_type=jnp.float32) + b2_ref[...]
        g_lo, g_hi = _unpack_bf16_pair(g_ref[...])
        cw = cut_ref[...]
        o_ref[0] = g_lo * w[:, :FH] * cw
        o_ref[1] = g_hi * w[:, FH:] * cw

    return pl.pallas_call(
        body,
        grid=(E_PAD // BE,),
        in_specs=[pl.BlockSpec((BE, NG), lambda i: (i, 0)),
                  pl.BlockSpec((BE, 1), lambda i: (i, 0)),
                  pl.BlockSpec((BE, HID // 2), lambda i: (i, 0)),
                  pl.BlockSpec((NG, HID), lambda i: (0, 0)),
                  pl.BlockSpec((1, HID), lambda i: (0, 0)),
                  pl.BlockSpec((HID, HID), lambda i: (0, 0)),
                  pl.BlockSpec((1, HID), lambda i: (0, 0))],
        out_specs=pl.BlockSpec((NC, BE, FH), lambda i: (0, i, 0)),
        out_shape=jax.ShapeDtypeStruct((NC, E_PAD, FH), jnp.float32),
    )(rbf, cut, g, w1t, b1, w2t, b2)


def _node_kernel(agg, h, w2t, b2, w3t, b3, wnt=None, bn=None):
    dual = wnt is not None

    def body(agg_ref, h_ref, w2_ref, b2_ref, w3_ref, b3_ref, *rest):
        if dual:
            wn_ref, bn_ref, hn_ref, hA_ref = rest
        else:
            (hn_ref,) = rest
        t = jnp.dot(agg_ref[...].astype(jnp.bfloat16), w2_ref[...],
                    preferred_element_type=jnp.float32) + b2_ref[...]
        t = _ssp(t)
        t = jnp.dot(t.astype(jnp.bfloat16), w3_ref[...],
                    preferred_element_type=jnp.float32) + b3_ref[...]
        hn = h_ref[...] + t
        hn_ref[...] = hn
        if dual:
            hA = jnp.dot(hn.astype(jnp.bfloat16), wn_ref[...],
                         preferred_element_type=jnp.float32) + bn_ref[...]
            hA_ref[...] = _pack_bf16_pair(hA)

    full = lambda i: (0, 0)
    blk = lambda i: (i, 0)
    in_specs = [pl.BlockSpec((BN, HID), blk), pl.BlockSpec((BN, HID), blk),
                pl.BlockSpec((HID, HID), full), pl.BlockSpec((1, HID), full),
                pl.BlockSpec((HID, HID), full), pl.BlockSpec((1, HID), full)]
    args = [agg, h, w2t, b2, w3t, b3]
    if dual:
        in_specs += [pl.BlockSpec((HID, HID), full), pl.BlockSpec((1, HID), full)]
        args += [wnt, bn]
        out_specs = [pl.BlockSpec((BN, HID), blk),
                     pl.BlockSpec((BN, HID // 2), blk)]
        out_shape = [jax.ShapeDtypeStruct((N_PAD, HID), jnp.float32),
                     jax.ShapeDtypeStruct((N_PAD, HID // 2), jnp.int32)]
    else:
        out_specs = pl.BlockSpec((BN, HID), blk)
        out_shape = jax.ShapeDtypeStruct((N_PAD, HID), jnp.float32)

    return pl.pallas_call(
        body,
        grid=(N_PAD // BN,),
        in_specs=in_specs,
        out_specs=out_specs,
        out_shape=out_shape,
    )(*args)


def _readout_kernel(h, batch_c, r1wt, r1b, r2w, r2b):
    def body(h_ref, b_ref, w1_ref, b1_ref, w2_ref, b2_ref, o_ref):
        t = jnp.dot(h_ref[...], w1_ref[...],
                    preferred_element_type=jnp.float32) + b1_ref[...]
        t = _ssp(t)
        e = jnp.sum(t * w2_ref[...], axis=1, keepdims=True) + b2_ref[...]
        onehot = (b_ref[...] == lax.broadcasted_iota(jnp.int32, (1, NMOL), 1)
                  ).astype(jnp.float32)
        mol = jnp.sum(onehot * e, axis=0, keepdims=True)

        @pl.when(pl.program_id(0) == 0)
        def _():
            o_ref[...] = jnp.zeros_like(o_ref)

        o_ref[...] += mol

    return pl.pallas_call(
        body,
        grid=(N_PAD // BN,),
        in_specs=[pl.BlockSpec((BN, HID), lambda i: (i, 0)),
                  pl.BlockSpec((BN, 1), lambda i: (i, 0)),
                  pl.BlockSpec((HID, HID // 2), lambda i: (0, 0)),
                  pl.BlockSpec((1, HID // 2), lambda i: (0, 0)),
                  pl.BlockSpec((1, HID // 2), lambda i: (0, 0)),
                  pl.BlockSpec((1, 1), lambda i: (0, 0))],
        out_specs=pl.BlockSpec((1, NMOL), lambda i: (0, 0)),
        out_shape=jax.ShapeDtypeStruct((1, NMOL), jnp.float32),
    )(h, batch_c, r1wt, r1b, r2w, r2b)


def kernel(z, edge_index, edge_attr, batch, emb, fw1, fb1, fw2, fb2,
           a1w, a1b, a2w, a2b, a3w, a3b, r1w, r1b, r2w, r2b):
    row = edge_index[0]
    col = edge_index[1]
    dist = edge_attr[:, 0]

    pad = E_PAD - N_EDGES
    spread = jnp.arange(pad, dtype=jnp.int32) % N_NODES
    col2d = jnp.concatenate([col, spread]).reshape(E_PAD // GB, GB)
    row2d = jnp.concatenate([row, spread]).reshape(E_PAD // GB, GB)
    dist_c = jnp.concatenate(
        [dist, jnp.full((pad,), 2.0 * CUT, dist.dtype)]).reshape(E_PAD, 1)

    emb_pad = jnp.zeros((128, HID), jnp.float32).at[: emb.shape[0]].set(emb)
    npad = N_PAD - N_NODES
    z_c = jnp.concatenate(
        [z.astype(jnp.int32), jnp.zeros((npad,), jnp.int32)]).reshape(N_PAD, 1)
    batch_c = jnp.concatenate(
        [batch.astype(jnp.int32),
         jnp.full((npad,), NMOL, jnp.int32)]).reshape(N_PAD, 1)

    fw1t = jnp.swapaxes(fw1, 1, 2).astype(jnp.bfloat16)
    fw2t = jnp.swapaxes(fw2, 1, 2).astype(jnp.bfloat16)
    a1wt = jnp.swapaxes(a1w, 1, 2).astype(jnp.bfloat16)
    a2wt = jnp.swapaxes(a2w, 1, 2).astype(jnp.bfloat16)
    a3wt = jnp.swapaxes(a3w, 1, 2).astype(jnp.bfloat16)
    fb1_2 = fb1[:, None, :]
    fb2_2 = fb2[:, None, :]
    a1b_2 = a1b[:, None, :]
    a2b_2 = a2b[:, None, :]
    a3b_2 = a3b[:, None, :]
    r1wt = r1w.T
    r1b_2 = r1b[None, :]
    r2b_2 = r2b[None, :]

    rbf, cut = _rbf_kernel(dist_c)
    h, hA = _emb_kernel(z_c, emb_pad, a1wt[0], a1b_2[0])

    for l in range(NL):
        g = _sc_gather(hA, col2d)
        msg = _filter_kernel(rbf, cut, g, fw1t[l], fb1_2[l], fw2t[l], fb2_2[l])
        agg = _sc_scatter_add(msg, row2d)
        if l < NL - 1:
            h, hA = _node_kernel(agg, h, a2wt[l], a2b_2[l], a3wt[l], a3b_2[l],
                                 a1wt[l + 1], a1b_2[l + 1])
        else:
            h = _node_kernel(agg, h, a2wt[l], a2b_2[l], a3wt[l], a3b_2[l])

    mol = _readout_kernel(h, batch_c, r1wt, r1b_2, r2w, r2b_2)
    return mol[0]


# speedup vs baseline: 3.1064x; 1.0614x over previous
import functools

import jax
import jax.numpy as jnp
import numpy as np
from jax import lax
from jax.experimental import pallas as pl
from jax.experimental.pallas import tpu as pltpu
from jax.experimental.pallas import tpu_sc as plsc

HID = 256
NG = 64
NL = 4
CUT = 5.0
NMOL = 64
N_NODES = 10000
N_PAD = 10240
N_EDGES = 160000
LOG2 = float(np.log(2.0))

NC = 2
NS = 16
NW = NC * NS

GB = 128
E_PAD = 163840
GNB = E_PAD // (NW * GB)
SNB = E_PAD // (NS * GB)
NPT = N_PAD // NS
FH = HID // NC


def _ssp(x):
    return jnp.maximum(x, 0.0) + jnp.log1p(jnp.exp(-jnp.abs(x))) - LOG2


def _pack_bf16_pair(x):
    lo = lax.bitcast_convert_type(
        x[:, :FH].astype(jnp.bfloat16), jnp.uint16).astype(jnp.int32)
    hi = lax.bitcast_convert_type(
        x[:, FH:].astype(jnp.bfloat16), jnp.uint16).astype(jnp.int32)
    return (hi << 16) | lo


def _unpack_bf16_pair(p):
    lo = lax.bitcast_convert_type(p << 16, jnp.float32)
    hi = lax.bitcast_convert_type(
        p & jnp.int32(np.uint32(0xFFFF0000)), jnp.float32)
    return lo, hi


def _sc_gather(table, idx2d):
    mesh = plsc.VectorSubcoreMesh(core_axis_name="c", subcore_axis_name="s")

    @functools.partial(
        pl.kernel,
        mesh=mesh,
        out_type=jax.ShapeDtypeStruct((E_PAD, HID // 2), jnp.int32),
        scratch_types=[
            pltpu.VMEM((GNB, GB), jnp.int32),
            pltpu.VMEM((GB, HID // 2), jnp.int32),
            pltpu.VMEM((GB, HID // 2), jnp.int32),
            pltpu.VMEM_SHARED((N_PAD, HID // 2), jnp.int32),
            pltpu.SemaphoreType.DMA,
            pltpu.SemaphoreType.DMA,
            pltpu.SemaphoreType.DMA,
            pltpu.SemaphoreType.DMA,
        ],
    )
    def k(table_hbm, idx_hbm, out_hbm, idx_v, buf0, buf1, tbl,
          gs0, gs1, ws0, ws1):
        wid = lax.axis_index("s") * NC + lax.axis_index("c")
        s = lax.axis_index("s")
        blk0 = wid * GNB
        pltpu.sync_copy(table_hbm.at[pl.ds(s * NPT, NPT)],
                        tbl.at[pl.ds(s * NPT, NPT)])
        pltpu.sync_copy(idx_hbm.at[pl.ds(blk0, GNB)], idx_v)
        plsc.subcore_barrier()

        def body(j2, carry):
            j0 = 2 * j2
            j1 = j0 + 1

            @pl.when(j2 > 0)
            def _():
                pltpu.make_async_copy(
                    buf0, out_hbm.at[pl.ds(0, GB)], ws0).wait()

            g0 = pltpu.async_copy(tbl.at[idx_v.at[j0]], buf0, gs0)

            @pl.when(j2 > 0)
            def _():
                pltpu.make_async_copy(
                    buf1, out_hbm.at[pl.ds(0, GB)], ws1).wait()

            g1 = pltpu.async_copy(tbl.at[idx_v.at[j1]], buf1, gs1)
            g0.wait()
            pltpu.async_copy(buf0, out_hbm.at[pl.ds((blk0 + j0) * GB, GB)], ws0)
            g1.wait()
            pltpu.async_copy(buf1, out_hbm.at[pl.ds((blk0 + j1) * GB, GB)], ws1)
            return carry

        lax.fori_loop(0, GNB // 2, body, 0)
        pltpu.make_async_copy(buf0, out_hbm.at[pl.ds(0, GB)], ws0).wait()
        pltpu.make_async_copy(buf1, out_hbm.at[pl.ds(0, GB)], ws1).wait()

    return k(table, idx2d)


def _sc_scatter_add(msg, idx2d):
    mesh = plsc.VectorSubcoreMesh(core_axis_name="c", subcore_axis_name="s")

    @functools.partial(
        pl.kernel,
        mesh=mesh,
        out_type=jax.ShapeDtypeStruct((N_PAD, HID), jnp.float32),
        scratch_types=[
            pltpu.VMEM((SNB, GB), jnp.int32),
            pltpu.VMEM((GB, FH), jnp.float32),
            pltpu.VMEM((GB, FH), jnp.float32),
            pltpu.VMEM_SHARED((N_PAD, FH), jnp.float32),
            pltpu.SemaphoreType.DMA,
            pltpu.SemaphoreType.DMA,
            pltpu.SemaphoreType.DMA,
            pltpu.SemaphoreType.DMA,
        ],
    )
    def k(msg_hbm, idx_hbm, out_hbm, idx_v, mb0, mb1, acc, ls0, ls1, ss0, ss1):
        c = lax.axis_index("c")
        s = lax.axis_index("s")

        def zrow(i, carry):
            for kk in range(FH // 16):
                mb0[i, pl.ds(kk * 16, 16)] = jnp.zeros((16,), jnp.float32)
            return carry

        lax.fori_loop(0, GB, zrow, 0)
        for t in range(NPT // GB):
            pltpu.sync_copy(mb0, acc.at[pl.ds(s * NPT + t * GB, GB)])
        plsc.subcore_barrier()

        pltpu.sync_copy(idx_hbm.at[pl.ds(s * SNB, SNB)], idx_v)

        def body(j2, carry):
            j0 = 2 * j2
            j1 = j0 + 1

            @pl.when(j2 > 0)
            def _():
                pltpu.make_async_copy(mb0, acc.at[pl.ds(0, GB)], ss0).wait()

            l0 = pltpu.async_copy(
                msg_hbm.at[c, pl.ds((s * SNB + j0) * GB, GB)], mb0, ls0)

            @pl.when(j2 > 0)
            def _():
                pltpu.make_async_copy(mb1, acc.at[pl.ds(0, GB)], ss1).wait()

            l1 = pltpu.async_copy(
                msg_hbm.at[c, pl.ds((s * SNB + j1) * GB, GB)], mb1, ls1)
            l0.wait()
            pltpu.async_copy(mb0, acc.at[idx_v.at[j0]], ss0, add=True)
            l1.wait()
            pltpu.async_copy(mb1, acc.at[idx_v.at[j1]], ss1, add=True)
            return carry

        lax.fori_loop(0, SNB // 2, body, 0)
        pltpu.make_async_copy(mb0, acc.at[pl.ds(0, GB)], ss0).wait()
        pltpu.make_async_copy(mb1, acc.at[pl.ds(0, GB)], ss1).wait()
        plsc.subcore_barrier()
        pltpu.sync_copy(acc.at[pl.ds(s * NPT, NPT)],
                        out_hbm.at[pl.ds(s * NPT, NPT), pl.ds(c * FH, FH)])

    return k(msg, idx2d)


BE = 4096
BN = 2048


def _rbf_kernel(dist_c):
    coeff = -0.5 / (CUT / NG) ** 2
    step = CUT / (NG - 1)

    def body(d_ref, rbf_ref, cut_ref):
        d = d_ref[...]
        offset = lax.broadcasted_iota(
            jnp.int32, (1, NG), 1).astype(jnp.float32) * step
        diff = d - offset
        rbf_ref[...] = jnp.exp(coeff * diff * diff)
        cut_ref[...] = 0.5 * (jnp.cos(d * (np.pi / CUT)) + 1.0) * (
            d < CUT).astype(jnp.float32)

    return pl.pallas_call(
        body,
        grid=(E_PAD // BE,),
        in_specs=[pl.BlockSpec((BE, 1), lambda i: (i, 0))],
        out_specs=[pl.BlockSpec((BE, NG), lambda i: (i, 0)),
                   pl.BlockSpec((BE, 1), lambda i: (i, 0))],
        out_shape=[jax.ShapeDtypeStruct((E_PAD, NG), jnp.float32),
                   jax.ShapeDtypeStruct((E_PAD, 1), jnp.float32)],
    )(dist_c)


def _emb_kernel(z_c, emb_pad, w1t, b1):
    def body(z_ref, emb_ref, w_ref, b_ref, h_ref, hA_ref):
        onehot = (z_ref[...] == lax.broadcasted_iota(jnp.int32, (1, 128), 1)
                  ).astype(jnp.float32)
        h = jnp.dot(onehot, emb_ref[...], preferred_element_type=jnp.float32)
        h_ref[...] = h
        hA = jnp.dot(h.astype(jnp.bfloat16), w_ref[...],
                     preferred_element_type=jnp.float32) + b_ref[...]
        hA_ref[...] = _pack_bf16_pair(hA)

    return pl.pallas_call(
        body,
        grid=(N_PAD // BN,),
        in_specs=[pl.BlockSpec((BN, 1), lambda i: (i, 0)),
                  pl.BlockSpec((128, HID), lambda i: (0, 0)),
                  pl.BlockSpec((HID, HID), lambda i: (0, 0)),
                  pl.BlockSpec((1, HID), lambda i: (0, 0))],
        out_specs=[pl.BlockSpec((BN, HID), lambda i: (i, 0)),
                   pl.BlockSpec((BN, HID // 2), lambda i: (i, 0))],
        out_shape=[jax.ShapeDtypeStruct((N_PAD, HID), jnp.float32),
                   jax.ShapeDtypeStruct((N_PAD, HID // 2), jnp.int32)],
    )(z_c, emb_pad, w1t, b1)


def _filter_kernel(rbf, cut, g, w1t, b1, w2t, b2):
    def body(rbf_ref, cut_ref, g_ref, w1_ref, b1_ref, w2_ref, b2_ref, o_ref):
        t = jnp.dot(rbf_ref[...].astype(jnp.bfloat16), w1_ref[...],
                    preferred_element_type=jnp.float32) + b1_ref[...]
        t = _ssp(t)
        w = jnp.dot(t.astype(jnp.bfloat16), w2_ref[...],
                    preferred_element_type=jnp.float32) + b2_ref[...]
        g_lo, g_hi = _unpack_bf16_pair(g_ref[...])
        cw = cut_ref[...]
        o_ref[0] = g_lo * w[:, :FH] * cw
        o_ref[1] = g_hi * w[:, FH:] * cw

    return pl.pallas_call(
        body,
        grid=(E_PAD // BE,),
        in_specs=[pl.BlockSpec((BE, NG), lambda i: (i, 0)),
                  pl.BlockSpec((BE, 1), lambda i: (i, 0)),
                  pl.BlockSpec((BE, HID // 2), lambda i: (i, 0)),
                  pl.BlockSpec((NG, HID), lambda i: (0, 0)),
                  pl.BlockSpec((1, HID), lambda i: (0, 0)),
                  pl.BlockSpec((HID, HID), lambda i: (0, 0)),
                  pl.BlockSpec((1, HID), lambda i: (0, 0))],
        out_specs=pl.BlockSpec((NC, BE, FH), lambda i: (0, i, 0)),
        out_shape=jax.ShapeDtypeStruct((NC, E_PAD, FH), jnp.float32),
    )(rbf, cut, g, w1t, b1, w2t, b2)


def _node_kernel(agg, h, w2t, b2, w3t, b3, wnt=None, bn=None):
    dual = wnt is not None

    def body(agg_ref, h_ref, w2_ref, b2_ref, w3_ref, b3_ref, *rest):
        if dual:
            wn_ref, bn_ref, hn_ref, hA_ref = rest
        else:
            (hn_ref,) = rest
        t = jnp.dot(agg_ref[...].astype(jnp.bfloat16), w2_ref[...],
                    preferred_element_type=jnp.float32) + b2_ref[...]
        t = _ssp(t)
        t = jnp.dot(t.astype(jnp.bfloat16), w3_ref[...],
                    preferred_element_type=jnp.float32) + b3_ref[...]
        hn = h_ref[...] + t
        hn_ref[...] = hn
        if dual:
            hA = jnp.dot(hn.astype(jnp.bfloat16), wn_ref[...],
                         preferred_element_type=jnp.float32) + bn_ref[...]
            hA_ref[...] = _pack_bf16_pair(hA)

    full = lambda i: (0, 0)
    blk = lambda i: (i, 0)
    in_specs = [pl.BlockSpec((BN, HID), blk), pl.BlockSpec((BN, HID), blk),
                pl.BlockSpec((HID, HID), full), pl.BlockSpec((1, HID), full),
                pl.BlockSpec((HID, HID), full), pl.BlockSpec((1, HID), full)]
    args = [agg, h, w2t, b2, w3t, b3]
    if dual:
        in_specs += [pl.BlockSpec((HID, HID), full), pl.BlockSpec((1, HID), full)]
        args += [wnt, bn]
        out_specs = [pl.BlockSpec((BN, HID), blk),
                     pl.BlockSpec((BN, HID // 2), blk)]
        out_shape = [jax.ShapeDtypeStruct((N_PAD, HID), jnp.float32),
                     jax.ShapeDtypeStruct((N_PAD, HID // 2), jnp.int32)]
    else:
        out_specs = pl.BlockSpec((BN, HID), blk)
        out_shape = jax.ShapeDtypeStruct((N_PAD, HID), jnp.float32)

    return pl.pallas_call(
        body,
        grid=(N_PAD // BN,),
        in_specs=in_specs,
        out_specs=out_specs,
        out_shape=out_shape,
    )(*args)


def _readout_kernel(h, batch_c, r1wt, r1b, r2w, r2b):
    def body(h_ref, b_ref, w1_ref, b1_ref, w2_ref, b2_ref, o_ref):
        t = jnp.dot(h_ref[...], w1_ref[...],
                    preferred_element_type=jnp.float32) + b1_ref[...]
        t = _ssp(t)
        e = jnp.sum(t * w2_ref[...], axis=1, keepdims=True) + b2_ref[...]
        onehot = (b_ref[...] == lax.broadcasted_iota(jnp.int32, (1, NMOL), 1)
                  ).astype(jnp.float32)
        mol = jnp.sum(onehot * e, axis=0, keepdims=True)

        @pl.when(pl.program_id(0) == 0)
        def _():
            o_ref[...] = jnp.zeros_like(o_ref)

        o_ref[...] += mol

    return pl.pallas_call(
        body,
        grid=(N_PAD // BN,),
        in_specs=[pl.BlockSpec((BN, HID), lambda i: (i, 0)),
                  pl.BlockSpec((BN, 1), lambda i: (i, 0)),
                  pl.BlockSpec((HID, HID // 2), lambda i: (0, 0)),
                  pl.BlockSpec((1, HID // 2), lambda i: (0, 0)),
                  pl.BlockSpec((1, HID // 2), lambda i: (0, 0)),
                  pl.BlockSpec((1, 1), lambda i: (0, 0))],
        out_specs=pl.BlockSpec((1, NMOL), lambda i: (0, 0)),
        out_shape=jax.ShapeDtypeStruct((1, NMOL), jnp.float32),
    )(h, batch_c, r1wt, r1b, r2w, r2b)


def kernel(z, edge_index, edge_attr, batch, emb, fw1, fb1, fw2, fb2,
           a1w, a1b, a2w, a2b, a3w, a3b, r1w, r1b, r2w, r2b):
    row = edge_index[0]
    col = edge_index[1]
    dist = edge_attr[:, 0]

    pad = E_PAD - N_EDGES
    spread = jnp.arange(pad, dtype=jnp.int32) % N_NODES
    col2d = jnp.concatenate([col, spread]).reshape(E_PAD // GB, GB)
    row2d = jnp.concatenate([row, spread]).reshape(E_PAD // GB, GB)
    dist_c = jnp.concatenate(
        [dist, jnp.full((pad,), 2.0 * CUT, dist.dtype)]).reshape(E_PAD, 1)

    emb_pad = jnp.zeros((128, HID), jnp.float32).at[: emb.shape[0]].set(emb)
    npad = N_PAD - N_NODES
    z_c = jnp.concatenate(
        [z.astype(jnp.int32), jnp.zeros((npad,), jnp.int32)]).reshape(N_PAD, 1)
    batch_c = jnp.concatenate(
        [batch.astype(jnp.int32),
         jnp.full((npad,), NMOL, jnp.int32)]).reshape(N_PAD, 1)

    fw1t = jnp.swapaxes(fw1, 1, 2).astype(jnp.bfloat16)
    fw2t = jnp.swapaxes(fw2, 1, 2).astype(jnp.bfloat16)
    a1wt = jnp.swapaxes(a1w, 1, 2).astype(jnp.bfloat16)
    a2wt = jnp.swapaxes(a2w, 1, 2).astype(jnp.bfloat16)
    a3wt = jnp.swapaxes(a3w, 1, 2).astype(jnp.bfloat16)
    fb1_2 = fb1[:, None, :]
    fb2_2 = fb2[:, None, :]
    a1b_2 = a1b[:, None, :]
    a2b_2 = a2b[:, None, :]
    a3b_2 = a3b[:, None, :]
    r1wt = r1w.T
    r1b_2 = r1b[None, :]
    r2b_2 = r2b[None, :]

    rbf, cut = _rbf_kernel(dist_c)
    h, hA = _emb_kernel(z_c, emb_pad, a1wt[0], a1b_2[0])

    for l in range(NL):
        g = _sc_gather(hA, col2d)
        msg = _filter_kernel(rbf, cut, g, fw1t[l], fb1_2[l], fw2t[l], fb2_2[l])
        agg = _sc_scatter_add(msg, row2d)
        if l < NL - 1:
            h, hA = _node_kernel(agg, h, a2wt[l], a2b_2[l], a3wt[l], a3b_2[l],
                                 a1wt[l + 1], a1b_2[l + 1])
        else:
            h = _node_kernel(agg, h, a2wt[l], a2b_2[l], a3wt[l], a3b_2[l])

    mol = _readout_kernel(h, batch_c, r1wt, r1b_2, r2w, r2b_2)
    return mol[0]


# speedup vs baseline: 3.1731x; 1.0215x over previous
import functools

import jax
import jax.numpy as jnp
import numpy as np
from jax import lax
from jax.experimental import pallas as pl
from jax.experimental.pallas import tpu as pltpu
from jax.experimental.pallas import tpu_sc as plsc

HID = 256
NG = 64
NL = 4
CUT = 5.0
NMOL = 64
N_NODES = 10000
N_PAD = 10240
N_EDGES = 160000
LOG2 = float(np.log(2.0))

NC = 2
NS = 16
NW = NC * NS

GB = 128
E_PAD = 163840
GNB = E_PAD // (NW * GB)
SNB = E_PAD // (NS * GB)
NPT = N_PAD // NS
FH = HID // NC


def _ssp(x):
    return jnp.maximum(x, 0.0) + jnp.log1p(jnp.exp(-jnp.abs(x))) - LOG2


def _pack_bf16_pair(x):
    lo = lax.bitcast_convert_type(
        x[:, :FH].astype(jnp.bfloat16), jnp.uint16).astype(jnp.int32)
    hi = lax.bitcast_convert_type(
        x[:, FH:].astype(jnp.bfloat16), jnp.uint16).astype(jnp.int32)
    return (hi << 16) | lo


def _unpack_bf16_pair(p):
    lo = lax.bitcast_convert_type(p << 16, jnp.float32)
    hi = lax.bitcast_convert_type(
        p & jnp.int32(np.uint32(0xFFFF0000)), jnp.float32)
    return lo, hi


def _sc_gather(table, idx2d):
    mesh = plsc.VectorSubcoreMesh(core_axis_name="c", subcore_axis_name="s")

    @functools.partial(
        pl.kernel,
        mesh=mesh,
        out_type=jax.ShapeDtypeStruct((E_PAD, HID // 2), jnp.int32),
        scratch_types=[
            pltpu.VMEM((GNB, GB), jnp.int32),
            pltpu.VMEM((GB, HID // 2), jnp.int32),
            pltpu.VMEM((GB, HID // 2), jnp.int32),
            pltpu.VMEM_SHARED((N_PAD, HID // 2), jnp.int32),
            pltpu.SemaphoreType.DMA,
            pltpu.SemaphoreType.DMA,
            pltpu.SemaphoreType.DMA,
            pltpu.SemaphoreType.DMA,
        ],
    )
    def k(table_hbm, idx_hbm, out_hbm, idx_v, buf0, buf1, tbl,
          gs0, gs1, ws0, ws1):
        wid = lax.axis_index("s") * NC + lax.axis_index("c")
        s = lax.axis_index("s")
        blk0 = wid * GNB
        pltpu.sync_copy(table_hbm.at[pl.ds(s * NPT, NPT)],
                        tbl.at[pl.ds(s * NPT, NPT)])
        pltpu.sync_copy(idx_hbm.at[pl.ds(blk0, GNB)], idx_v)
        plsc.subcore_barrier()

        def body(j2, carry):
            j0 = 2 * j2
            j1 = j0 + 1

            @pl.when(j2 > 0)
            def _():
                pltpu.make_async_copy(
                    buf0, out_hbm.at[pl.ds(0, GB)], ws0).wait()

            g0 = pltpu.async_copy(tbl.at[idx_v.at[j0]], buf0, gs0)

            @pl.when(j2 > 0)
            def _():
                pltpu.make_async_copy(
                    buf1, out_hbm.at[pl.ds(0, GB)], ws1).wait()

            g1 = pltpu.async_copy(tbl.at[idx_v.at[j1]], buf1, gs1)
            g0.wait()
            pltpu.async_copy(buf0, out_hbm.at[pl.ds((blk0 + j0) * GB, GB)], ws0)
            g1.wait()
            pltpu.async_copy(buf1, out_hbm.at[pl.ds((blk0 + j1) * GB, GB)], ws1)
            return carry

        lax.fori_loop(0, GNB // 2, body, 0)
        pltpu.make_async_copy(buf0, out_hbm.at[pl.ds(0, GB)], ws0).wait()
        pltpu.make_async_copy(buf1, out_hbm.at[pl.ds(0, GB)], ws1).wait()

    return k(table, idx2d)


def _sc_scatter_add(msg, idx2d):
    mesh = plsc.VectorSubcoreMesh(core_axis_name="c", subcore_axis_name="s")

    @functools.partial(
        pl.kernel,
        mesh=mesh,
        out_type=jax.ShapeDtypeStruct((N_PAD, HID), jnp.float32),
        scratch_types=[
            pltpu.VMEM((SNB, GB), jnp.int32),
            pltpu.VMEM((GB, FH), jnp.float32),
            pltpu.VMEM((GB, FH), jnp.float32),
            pltpu.VMEM_SHARED((N_PAD, FH), jnp.float32),
            pltpu.SemaphoreType.DMA,
            pltpu.SemaphoreType.DMA,
            pltpu.SemaphoreType.DMA,
            pltpu.SemaphoreType.DMA,
        ],
    )
    def k(msg_hbm, idx_hbm, out_hbm, idx_v, mb0, mb1, acc, ls0, ls1, ss0, ss1):
        c = lax.axis_index("c")
        s = lax.axis_index("s")

        def zrow(i, carry):
            for kk in range(FH // 16):
                mb0[i, pl.ds(kk * 16, 16)] = jnp.zeros((16,), jnp.float32)
            return carry

        lax.fori_loop(0, GB, zrow, 0)
        for t in range(NPT // GB):
            pltpu.sync_copy(mb0, acc.at[pl.ds(s * NPT + t * GB, GB)])
        plsc.subcore_barrier()

        pltpu.sync_copy(idx_hbm.at[pl.ds(s * SNB, SNB)], idx_v)

        def body(j2, carry):
            j0 = 2 * j2
            j1 = j0 + 1

            @pl.when(j2 > 0)
            def _():
                pltpu.make_async_copy(mb0, acc.at[pl.ds(0, GB)], ss0).wait()

            l0 = pltpu.async_copy(
                msg_hbm.at[c, pl.ds((s * SNB + j0) * GB, GB)], mb0, ls0)

            @pl.when(j2 > 0)
            def _():
                pltpu.make_async_copy(mb1, acc.at[pl.ds(0, GB)], ss1).wait()

            l1 = pltpu.async_copy(
                msg_hbm.at[c, pl.ds((s * SNB + j1) * GB, GB)], mb1, ls1)
            l0.wait()
            pltpu.async_copy(mb0, acc.at[idx_v.at[j0]], ss0, add=True)
            l1.wait()
            pltpu.async_copy(mb1, acc.at[idx_v.at[j1]], ss1, add=True)
            return carry

        lax.fori_loop(0, SNB // 2, body, 0)
        pltpu.make_async_copy(mb0, acc.at[pl.ds(0, GB)], ss0).wait()
        pltpu.make_async_copy(mb1, acc.at[pl.ds(0, GB)], ss1).wait()
        plsc.subcore_barrier()
        pltpu.sync_copy(acc.at[pl.ds(s * NPT, NPT)],
                        out_hbm.at[pl.ds(s * NPT, NPT), pl.ds(c * FH, FH)])

    return k(msg, idx2d)


BE = 8192
BN = 2048


def _rbf_kernel(dist_c):
    coeff = -0.5 / (CUT / NG) ** 2
    step = CUT / (NG - 1)

    def body(d_ref, rbf_ref, cut_ref):
        d = d_ref[...]
        offset = lax.broadcasted_iota(
            jnp.int32, (1, NG), 1).astype(jnp.float32) * step
        diff = d - offset
        rbf_ref[...] = jnp.exp(coeff * diff * diff)
        cut_ref[...] = 0.5 * (jnp.cos(d * (np.pi / CUT)) + 1.0) * (
            d < CUT).astype(jnp.float32)

    return pl.pallas_call(
        body,
        grid=(E_PAD // BE,),
        in_specs=[pl.BlockSpec((BE, 1), lambda i: (i, 0))],
        out_specs=[pl.BlockSpec((BE, NG), lambda i: (i, 0)),
                   pl.BlockSpec((BE, 1), lambda i: (i, 0))],
        out_shape=[jax.ShapeDtypeStruct((E_PAD, NG), jnp.float32),
                   jax.ShapeDtypeStruct((E_PAD, 1), jnp.float32)],
    )(dist_c)


def _emb_kernel(z_c, emb_pad, w1t, b1):
    def body(z_ref, emb_ref, w_ref, b_ref, h_ref, hA_ref):
        onehot = (z_ref[...] == lax.broadcasted_iota(jnp.int32, (1, 128), 1)
                  ).astype(jnp.float32)
        h = jnp.dot(onehot, emb_ref[...], preferred_element_type=jnp.float32)
        h_ref[...] = h
        hA = jnp.dot(h.astype(jnp.bfloat16), w_ref[...],
                     preferred_element_type=jnp.float32) + b_ref[...]
        hA_ref[...] = _pack_bf16_pair(hA)

    return pl.pallas_call(
        body,
        grid=(N_PAD // BN,),
        in_specs=[pl.BlockSpec((BN, 1), lambda i: (i, 0)),
                  pl.BlockSpec((128, HID), lambda i: (0, 0)),
                  pl.BlockSpec((HID, HID), lambda i: (0, 0)),
                  pl.BlockSpec((1, HID), lambda i: (0, 0))],
        out_specs=[pl.BlockSpec((BN, HID), lambda i: (i, 0)),
                   pl.BlockSpec((BN, HID // 2), lambda i: (i, 0))],
        out_shape=[jax.ShapeDtypeStruct((N_PAD, HID), jnp.float32),
                   jax.ShapeDtypeStruct((N_PAD, HID // 2), jnp.int32)],
    )(z_c, emb_pad, w1t, b1)


def _filter_kernel(rbf, cut, g, w1t, b1, w2t, b2):
    def body(rbf_ref, cut_ref, g_ref, w1_ref, b1_ref, w2_ref, b2_ref, o_ref):
        t = jnp.dot(rbf_ref[...].astype(jnp.bfloat16), w1_ref[...],
                    preferred_element_type=jnp.float32) + b1_ref[...]
        t = _ssp(t)
        w = jnp.dot(t.astype(jnp.bfloat16), w2_ref[...],
                    preferred_element_type=jnp.float32) + b2_ref[...]
        g_lo, g_hi = _unpack_bf16_pair(g_ref[...])
        cw = cut_ref[...]
        o_ref[0] = g_lo * w[:, :FH] * cw
        o_ref[1] = g_hi * w[:, FH:] * cw

    return pl.pallas_call(
        body,
        grid=(E_PAD // BE,),
        in_specs=[pl.BlockSpec((BE, NG), lambda i: (i, 0)),
                  pl.BlockSpec((BE, 1), lambda i: (i, 0)),
                  pl.BlockSpec((BE, HID // 2), lambda i: (i, 0)),
                  pl.BlockSpec((NG, HID), lambda i: (0, 0)),
                  pl.BlockSpec((1, HID), lambda i: (0, 0)),
                  pl.BlockSpec((HID, HID), lambda i: (0, 0)),
                  pl.BlockSpec((1, HID), lambda i: (0, 0))],
        out_specs=pl.BlockSpec((NC, BE, FH), lambda i: (0, i, 0)),
        out_shape=jax.ShapeDtypeStruct((NC, E_PAD, FH), jnp.float32),
    )(rbf, cut, g, w1t, b1, w2t, b2)


def _node_kernel(agg, h, w2t, b2, w3t, b3, wnt=None, bn=None):
    dual = wnt is not None

    def body(agg_ref, h_ref, w2_ref, b2_ref, w3_ref, b3_ref, *rest):
        if dual:
            wn_ref, bn_ref, hn_ref, hA_ref = rest
        else:
            (hn_ref,) = rest
        t = jnp.dot(agg_ref[...].astype(jnp.bfloat16), w2_ref[...],
                    preferred_element_type=jnp.float32) + b2_ref[...]
        t = _ssp(t)
        t = jnp.dot(t.astype(jnp.bfloat16), w3_ref[...],
                    preferred_element_type=jnp.float32) + b3_ref[...]
        hn = h_ref[...] + t
        hn_ref[...] = hn
        if dual:
            hA = jnp.dot(hn.astype(jnp.bfloat16), wn_ref[...],
                         preferred_element_type=jnp.float32) + bn_ref[...]
            hA_ref[...] = _pack_bf16_pair(hA)

    full = lambda i: (0, 0)
    blk = lambda i: (i, 0)
    in_specs = [pl.BlockSpec((BN, HID), blk), pl.BlockSpec((BN, HID), blk),
                pl.BlockSpec((HID, HID), full), pl.BlockSpec((1, HID), full),
                pl.BlockSpec((HID, HID), full), pl.BlockSpec((1, HID), full)]
    args = [agg, h, w2t, b2, w3t, b3]
    if dual:
        in_specs += [pl.BlockSpec((HID, HID), full), pl.BlockSpec((1, HID), full)]
        args += [wnt, bn]
        out_specs = [pl.BlockSpec((BN, HID), blk),
                     pl.BlockSpec((BN, HID // 2), blk)]
        out_shape = [jax.ShapeDtypeStruct((N_PAD, HID), jnp.float32),
                     jax.ShapeDtypeStruct((N_PAD, HID // 2), jnp.int32)]
    else:
        out_specs = pl.BlockSpec((BN, HID), blk)
        out_shape = jax.ShapeDtypeStruct((N_PAD, HID), jnp.float32)

    return pl.pallas_call(
        body,
        grid=(N_PAD // BN,),
        in_specs=in_specs,
        out_specs=out_specs,
        out_shape=out_shape,
    )(*args)


def _readout_kernel(h, batch_c, r1wt, r1b, r2w, r2b):
    def body(h_ref, b_ref, w1_ref, b1_ref, w2_ref, b2_ref, o_ref):
        t = jnp.dot(h_ref[...], w1_ref[...],
                    preferred_element_type=jnp.float32) + b1_ref[...]
        t = _ssp(t)
        e = jnp.sum(t * w2_ref[...], axis=1, keepdims=True) + b2_ref[...]
        onehot = (b_ref[...] == lax.broadcasted_iota(jnp.int32, (1, NMOL), 1)
                  ).astype(jnp.float32)
        mol = jnp.sum(onehot * e, axis=0, keepdims=True)

        @pl.when(pl.program_id(0) == 0)
        def _():
            o_ref[...] = jnp.zeros_like(o_ref)

        o_ref[...] += mol

    return pl.pallas_call(
        body,
        grid=(N_PAD // BN,),
        in_specs=[pl.BlockSpec((BN, HID), lambda i: (i, 0)),
                  pl.BlockSpec((BN, 1), lambda i: (i, 0)),
                  pl.BlockSpec((HID, HID // 2), lambda i: (0, 0)),
                  pl.BlockSpec((1, HID // 2), lambda i: (0, 0)),
                  pl.BlockSpec((1, HID // 2), lambda i: (0, 0)),
                  pl.BlockSpec((1, 1), lambda i: (0, 0))],
        out_specs=pl.BlockSpec((1, NMOL), lambda i: (0, 0)),
        out_shape=jax.ShapeDtypeStruct((1, NMOL), jnp.float32),
    )(h, batch_c, r1wt, r1b, r2w, r2b)


def kernel(z, edge_index, edge_attr, batch, emb, fw1, fb1, fw2, fb2,
           a1w, a1b, a2w, a2b, a3w, a3b, r1w, r1b, r2w, r2b):
    row = edge_index[0]
    col = edge_index[1]
    dist = edge_attr[:, 0]

    pad = E_PAD - N_EDGES
    spread = jnp.arange(pad, dtype=jnp.int32) % N_NODES
    col2d = jnp.concatenate([col, spread]).reshape(E_PAD // GB, GB)
    row2d = jnp.concatenate([row, spread]).reshape(E_PAD // GB, GB)
    dist_c = jnp.concatenate(
        [dist, jnp.full((pad,), 2.0 * CUT, dist.dtype)]).reshape(E_PAD, 1)

    emb_pad = jnp.zeros((128, HID), jnp.float32).at[: emb.shape[0]].set(emb)
    npad = N_PAD - N_NODES
    z_c = jnp.concatenate(
        [z.astype(jnp.int32), jnp.zeros((npad,), jnp.int32)]).reshape(N_PAD, 1)
    batch_c = jnp.concatenate(
        [batch.astype(jnp.int32),
         jnp.full((npad,), NMOL, jnp.int32)]).reshape(N_PAD, 1)

    fw1t = jnp.swapaxes(fw1, 1, 2).astype(jnp.bfloat16)
    fw2t = jnp.swapaxes(fw2, 1, 2).astype(jnp.bfloat16)
    a1wt = jnp.swapaxes(a1w, 1, 2).astype(jnp.bfloat16)
    a2wt = jnp.swapaxes(a2w, 1, 2).astype(jnp.bfloat16)
    a3wt = jnp.swapaxes(a3w, 1, 2).astype(jnp.bfloat16)
    fb1_2 = fb1[:, None, :]
    fb2_2 = fb2[:, None, :]
    a1b_2 = a1b[:, None, :]
    a2b_2 = a2b[:, None, :]
    a3b_2 = a3b[:, None, :]
    r1wt = r1w.T
    r1b_2 = r1b[None, :]
    r2b_2 = r2b[None, :]

    rbf, cut = _rbf_kernel(dist_c)
    h, hA = _emb_kernel(z_c, emb_pad, a1wt[0], a1b_2[0])

    for l in range(NL):
        g = _sc_gather(hA, col2d)
        msg = _filter_kernel(rbf, cut, g, fw1t[l], fb1_2[l], fw2t[l], fb2_2[l])
        agg = _sc_scatter_add(msg, row2d)
        if l < NL - 1:
            h, hA = _node_kernel(agg, h, a2wt[l], a2b_2[l], a3wt[l], a3b_2[l],
                                 a1wt[l + 1], a1b_2[l + 1])
        else:
            h = _node_kernel(agg, h, a2wt[l], a2b_2[l], a3wt[l], a3b_2[l])

    mol = _readout_kernel(h, batch_c, r1wt, r1b_2, r2w, r2b_2)
    return mol[0]
